# Initial kernel scaffold; baseline (speedup 1.0000x reference)
import jax
import jax.numpy as jnp
from jax.experimental import pallas as pl


def kernel(loc_data, conf_data, priors, targets, im_names):
    raise NotImplementedError("write your pallas kernel here")


# speedup vs baseline: 19.3015x; 19.3015x over previous
import functools

import jax
import jax.numpy as jnp
from jax.experimental import pallas as pl
from jax.experimental.pallas import tpu as pltpu

VAR0 = 0.1
VAR1 = 0.2
K = 5
T1 = 0.35
T2 = 0.5
ALPHA = 0.25
GAMMA = 2.0
BETA = 0.11


def _hamloss_body(loc_ref, conf_ref, pri_ref, tgtT_ref,
                  tx1_ref, ty1_ref, tx2_ref, ty2_ref, tlab_ref,
                  out_ref, iou_s, bts_s, bti_s, cbpi_s, cps_s,
                  *, n_obj, p_real, p_pad):
    f32 = jnp.float32
    i32 = jnp.int32

    loc = loc_ref[0]
    pcx = pri_ref[0:1, :]
    pcy = pri_ref[1:2, :]
    pw = pri_ref[2:3, :]
    ph = pri_ref[3:4, :]
    tgtT = tgtT_ref[0]
    tx1 = tx1_ref[0]
    ty1 = ty1_ref[0]
    tx2 = tx2_ref[0]
    ty2 = ty2_ref[0]

    colI = jax.lax.broadcasted_iota(i32, (n_obj, p_pad), 1)
    rowI = jax.lax.broadcasted_iota(i32, (n_obj, p_pad), 0)
    col1 = jax.lax.broadcasted_iota(i32, (1, p_pad), 1)
    rowV = jax.lax.broadcasted_iota(i32, (n_obj, 1), 0)
    valid = col1 < p_real

    area_t = (tx2 - tx1) * (ty2 - ty1)

    def jaccard(bx1, by1, bx2, by2, area_b):
        ltx = jnp.maximum(tx1, bx1)
        lty = jnp.maximum(ty1, by1)
        rbx = jnp.minimum(tx2, bx2)
        rby = jnp.minimum(ty2, by2)
        iw = jnp.clip(rbx - ltx, 0.0, None)
        ih = jnp.clip(rby - lty, 0.0, None)
        inter = iw * ih
        return inter / (area_t + area_b - inter)

    px1 = pcx - pw * 0.5
    py1 = pcy - ph * 0.5
    px2 = pcx + pw * 0.5
    py2 = pcy + ph * 0.5
    iou0 = jaccard(px1, py1, px2, py2, pw * ph)
    iou0 = jnp.where(valid, iou0, -1.0)
    iou_s[...] = iou0

    bts0 = jnp.max(iou0, axis=0, keepdims=True)
    bti0 = jnp.min(jnp.where(iou0 == bts0, rowI, n_obj), axis=0, keepdims=True)
    bts_s[...] = bts0
    bti_s[...] = bti0

    def greedy_body(_, carry):
        iou = iou_s[...]
        bps = jnp.max(iou, axis=1, keepdims=True)
        bpi = jnp.min(jnp.where(iou == bps, colI, p_pad), axis=1, keepdims=True)
        val = jnp.max(bps)
        j = jnp.min(jnp.where(bps == val, rowV, n_obj))
        i = jnp.sum(jnp.where(rowV == j, bpi, 0))
        iou_s[...] = jnp.where((rowI == j) | (colI == i), -1.0, iou)
        mi = col1 == i
        bts_s[...] = jnp.where(mi, val, bts_s[...])
        bti_s[...] = jnp.where(mi, j, bti_s[...])
        return carry

    jax.lax.fori_loop(0, n_obj, greedy_body, 0)

    bts = bts_s[...]
    bti = bti_s[...]

    def gather_rows(idx_row):
        oh = (idx_row == rowI).astype(f32)
        return jax.lax.dot_general(tgtT, oh, (((1,), (0,)), ((), ())),
                                   preferred_element_type=f32)

    g1 = gather_rows(bti)
    conf1 = jnp.where(bts < T1, 0.0, g1[4:5, :])

    def encode(g):
        mx1 = g[0:1, :]
        my1 = g[1:2, :]
        mx2 = g[2:3, :]
        my2 = g[3:4, :]
        e0 = ((mx1 + mx2) * 0.5 - pcx) / (VAR0 * pw)
        e1 = ((my1 + my2) * 0.5 - pcy) / (VAR0 * ph)
        e2 = jnp.log(jnp.clip((mx2 - mx1) / pw, 1e-8, None)) / VAR1
        e3 = jnp.log(jnp.clip((my2 - my1) / ph, 1e-8, None)) / VAR1
        return e0, e1, e2, e3

    enc1 = encode(g1)

    dl0 = loc[0:1, :]
    dl1 = loc[1:2, :]
    dl2 = loc[2:3, :]
    dl3 = loc[3:4, :]
    dcx = pcx + dl0 * (VAR0) * pw
    dcy = pcy + dl1 * (VAR0) * ph
    dw = pw * jnp.exp(dl2 * VAR1)
    dh = ph * jnp.exp(dl3 * VAR1)
    dx1 = dcx - dw * 0.5
    dy1 = dcy - dh * 0.5
    dx2 = dx1 + dw
    dy2 = dy1 + dh
    c_iou = jaccard(dx1, dy1, dx2, dy2, dw * dh)
    c_iou = jnp.where(valid, c_iou, 0.0)

    cbps = jnp.max(c_iou, axis=0, keepdims=True)
    cbpi = jnp.min(jnp.where(c_iou == cbps, rowI, n_obj), axis=0, keepdims=True)

    iou_s[...] = c_iou * (c_iou >= T2).astype(f32)

    ords = []
    tscs = []
    for _ in range(K):
        cm = iou_s[...]
        tk = jnp.max(cm, axis=1, keepdims=True)
        ok = jnp.min(jnp.where(cm == tk, colI, p_pad), axis=1, keepdims=True)
        iou_s[...] = jnp.where(colI == ok, -1.0, cm)
        ords.append(ok)
        tscs.append(tk)

    cgs = [jnp.sum(jnp.where(colI == ok, conf1, 0.0), axis=1, keepdims=True)
           for ok in ords]

    cbpi_s[...] = cbpi
    cps_s[...] = jnp.zeros((1, p_pad), f32)

    def scatter_body(i, carry):
        for k in range(K):
            p = jnp.sum(jnp.where(rowV == i, ords[k], 0))
            ts = jnp.sum(jnp.where(rowV == i, tscs[k], 0.0))
            cg = jnp.sum(jnp.where(rowV == i, cgs[k], 0.0))
            hit = (cg < 1.0) & (ts > 0.0)
            m = (col1 == p) & hit
            cbpi_s[...] = jnp.where(m, i, cbpi_s[...])
            cps_s[...] = jnp.where(m, ts, cps_s[...])
        return carry

    jax.lax.fori_loop(0, n_obj, scatter_body, 0)

    cps = cps_s[...]
    g2 = gather_rows(cbpi_s[...])
    conf2 = jnp.where(cps < T2, -1.0, g2[4:5, :])
    enc2 = encode(g2)

    ign = (bts < T1) & (~(cbps < T2)) & (cps < T2)
    conf1 = jnp.where(ign, -1.0, conf1)

    validf = valid.astype(f32)
    m1 = ((conf1 > 0) & valid).astype(f32)
    m2 = ((conf2 > 0) & valid).astype(f32)
    n1 = jnp.sum(m1)
    n2 = jnp.sum(m2)

    def smooth_l1(enc, m):
        s = jnp.zeros((), f32)
        for r in range(4):
            x = jnp.abs(loc[r:r + 1, :] - enc[r])
            l = jnp.where(x >= BETA, x - 0.5 * BETA, 0.5 * x * x / BETA)
            s = s + jnp.sum(l * m)
        return s

    sl1 = smooth_l1(enc1, m1)
    sl2 = smooth_l1(enc2, m2)

    c0 = conf_ref[0][0:1, :]
    c1 = conf_ref[0][1:2, :]

    def focal(t_row, fiou):
        keep = ((t_row >= 0) & valid).astype(f32)
        t = jnp.maximum(t_row, 0.0)
        x = jnp.where(t >= 0.5, c1, c0)
        ce = jnp.maximum(x, 0.0) - x * t + jnp.log1p(jnp.exp(-jnp.abs(x)))
        a = t * ALPHA + (1.0 - t) * (1.0 - ALPHA)
        if fiou is not None:
            a = a * fiou
        sig = 1.0 / (1.0 + jnp.exp(-x))
        pt = jnp.where(t == 1.0, sig, 1.0 - sig)
        om = 1.0 - pt
        return jnp.sum(a * om * om * ce * keep)

    f1 = focal(conf1, None)
    f2 = focal(conf2, cps)

    lane = jax.lax.broadcasted_iota(i32, (1, 128), 1)
    outv = (jnp.where(lane == 0, n1, 0.0) + jnp.where(lane == 1, n2, 0.0)
            + jnp.where(lane == 2, sl1, 0.0) + jnp.where(lane == 3, sl2, 0.0)
            + jnp.where(lane == 4, f1, 0.0) + jnp.where(lane == 5, f2, 0.0))
    out_ref[...] = outv[None]


def kernel(loc_data, conf_data, priors, targets, im_names):
    B, P, _ = loc_data.shape
    n_obj = targets.shape[1]
    p_pad = ((P + 127) // 128) * 128

    locT = jnp.pad(jnp.transpose(loc_data, (0, 2, 1)),
                   ((0, 0), (0, 0), (0, p_pad - P)))
    confT = jnp.pad(jnp.transpose(conf_data, (0, 2, 1)),
                    ((0, 0), (0, 0), (0, p_pad - P)))
    priT = jnp.transpose(priors, (1, 0))
    pad_col = jnp.array([5.0, 5.0, 0.1, 0.1], jnp.float32)[:, None]
    priT = jnp.concatenate(
        [priT, jnp.broadcast_to(pad_col, (4, p_pad - P))], axis=1)
    tgtT = jnp.transpose(targets, (0, 2, 1))
    tx1 = targets[:, :, 0:1]
    ty1 = targets[:, :, 1:2]
    tx2 = targets[:, :, 2:3]
    ty2 = targets[:, :, 3:4]
    tlab = targets[:, :, 4:5]

    body = functools.partial(_hamloss_body, n_obj=n_obj, p_real=P,
                             p_pad=p_pad)
    out = pl.pallas_call(
        body,
        grid=(B,),
        in_specs=[
            pl.BlockSpec((1, 4, p_pad), lambda b: (b, 0, 0)),
            pl.BlockSpec((1, 2, p_pad), lambda b: (b, 0, 0)),
            pl.BlockSpec((4, p_pad), lambda b: (0, 0)),
            pl.BlockSpec((1, 5, n_obj), lambda b: (b, 0, 0)),
            pl.BlockSpec((1, n_obj, 1), lambda b: (b, 0, 0)),
            pl.BlockSpec((1, n_obj, 1), lambda b: (b, 0, 0)),
            pl.BlockSpec((1, n_obj, 1), lambda b: (b, 0, 0)),
            pl.BlockSpec((1, n_obj, 1), lambda b: (b, 0, 0)),
            pl.BlockSpec((1, n_obj, 1), lambda b: (b, 0, 0)),
        ],
        out_specs=pl.BlockSpec((1, 1, 128), lambda b: (b, 0, 0)),
        out_shape=jax.ShapeDtypeStruct((B, 1, 128), jnp.float32),
        scratch_shapes=[
            pltpu.VMEM((n_obj, p_pad), jnp.float32),
            pltpu.VMEM((1, p_pad), jnp.float32),
            pltpu.VMEM((1, p_pad), jnp.int32),
            pltpu.VMEM((1, p_pad), jnp.int32),
            pltpu.VMEM((1, p_pad), jnp.float32),
        ],
        compiler_params=pltpu.CompilerParams(
            dimension_semantics=("arbitrary",)),
    )(locT, confT, priT, tgtT, tx1, ty1, tx2, ty2, tlab)

    n1 = jnp.sum(out[:, 0, 0])
    n2 = jnp.sum(out[:, 0, 1])
    sl1 = jnp.sum(out[:, 0, 2])
    sl2 = jnp.sum(out[:, 0, 3])
    f1 = jnp.sum(out[:, 0, 4])
    f2 = jnp.sum(out[:, 0, 5])

    has1 = n1 > 0
    has2 = n2 > 0
    ll1 = sl1 / n1
    ll2 = sl2 / n2
    cl1 = f1 / n1
    cl2 = f2 / n2
    fallback = jnp.asarray(0.0001, jnp.float32)
    loc_loss = jnp.where(has1 & (~has2), ll1,
               jnp.where(has1 & has2, ll1 + ll2,
               jnp.where((~has1) & has2, ll2, fallback)))
    cls_loss = jnp.where(has1 & (~has2), cl1,
               jnp.where(has1 & has2, cl1 + cl2,
               jnp.where((~has1) & has2, cl2, fallback)))
    return (loc_loss, cls_loss)


# speedup vs baseline: 19.3214x; 1.0010x over previous
import functools

import jax
import jax.numpy as jnp
from jax.experimental import pallas as pl
from jax.experimental.pallas import tpu as pltpu

VAR0 = 0.1
VAR1 = 0.2
K = 5
T1 = 0.35
T2 = 0.5
ALPHA = 0.25
GAMMA = 2.0
BETA = 0.11


def _hamloss_body(loc_ref, conf_ref, pri_ref, tgtT_ref,
                  tx1_ref, ty1_ref, tx2_ref, ty2_ref, tlab_ref,
                  out_ref, iou_s, bts_s, bti_s, cbpi_s, cps_s,
                  *, n_obj, p_real, p_pad):
    f32 = jnp.float32
    i32 = jnp.int32

    loc = loc_ref[0]
    pcx = pri_ref[0:1, :]
    pcy = pri_ref[1:2, :]
    pw = pri_ref[2:3, :]
    ph = pri_ref[3:4, :]
    tgtT = tgtT_ref[0]
    tx1 = tx1_ref[0]
    ty1 = ty1_ref[0]
    tx2 = tx2_ref[0]
    ty2 = ty2_ref[0]

    colI = jax.lax.broadcasted_iota(i32, (n_obj, p_pad), 1)
    rowI = jax.lax.broadcasted_iota(i32, (n_obj, p_pad), 0)
    col1 = jax.lax.broadcasted_iota(i32, (1, p_pad), 1)
    rowV = jax.lax.broadcasted_iota(i32, (n_obj, 1), 0)
    valid = col1 < p_real

    area_t = (tx2 - tx1) * (ty2 - ty1)

    def jaccard(bx1, by1, bx2, by2, area_b):
        ltx = jnp.maximum(tx1, bx1)
        lty = jnp.maximum(ty1, by1)
        rbx = jnp.minimum(tx2, bx2)
        rby = jnp.minimum(ty2, by2)
        iw = jnp.clip(rbx - ltx, 0.0, None)
        ih = jnp.clip(rby - lty, 0.0, None)
        inter = iw * ih
        return inter / (area_t + area_b - inter)

    px1 = pcx - pw * 0.5
    py1 = pcy - ph * 0.5
    px2 = pcx + pw * 0.5
    py2 = pcy + ph * 0.5
    iou0 = jaccard(px1, py1, px2, py2, pw * ph)
    iou0 = jnp.where(valid, iou0, -1.0)
    iou_s[...] = iou0

    bts0 = jnp.max(iou0, axis=0, keepdims=True)
    bti0 = jnp.min(jnp.where(iou0 == bts0, rowI, n_obj), axis=0, keepdims=True)
    bts_s[...] = bts0
    bti_s[...] = bti0

    def greedy_body(_, carry):
        iou = iou_s[...]
        bps = jnp.max(iou, axis=1, keepdims=True)
        bpi = jnp.min(jnp.where(iou == bps, colI, p_pad), axis=1, keepdims=True)
        val = jnp.max(bps)
        j = jnp.min(jnp.where(bps == val, rowV, n_obj))
        i = jnp.sum(jnp.where(rowV == j, bpi, 0))
        iou_s[...] = jnp.where((rowI == j) | (colI == i), -1.0, iou)
        mi = col1 == i
        bts_s[...] = jnp.where(mi, val, bts_s[...])
        bti_s[...] = jnp.where(mi, j, bti_s[...])
        return carry

    jax.lax.fori_loop(0, n_obj, greedy_body, 0)

    bts = bts_s[...]
    bti = bti_s[...]

    def gather_rows(idx_row):
        oh = (idx_row == rowI).astype(f32)
        return jax.lax.dot_general(tgtT, oh, (((1,), (0,)), ((), ())),
                                   preferred_element_type=f32)

    g1 = gather_rows(bti)
    conf1 = jnp.where(bts < T1, 0.0, g1[4:5, :])

    def encode(g):
        mx1 = g[0:1, :]
        my1 = g[1:2, :]
        mx2 = g[2:3, :]
        my2 = g[3:4, :]
        e0 = ((mx1 + mx2) * 0.5 - pcx) / (VAR0 * pw)
        e1 = ((my1 + my2) * 0.5 - pcy) / (VAR0 * ph)
        e2 = jnp.log(jnp.clip((mx2 - mx1) / pw, 1e-8, None)) / VAR1
        e3 = jnp.log(jnp.clip((my2 - my1) / ph, 1e-8, None)) / VAR1
        return e0, e1, e2, e3

    enc1 = encode(g1)

    dl0 = loc[0:1, :]
    dl1 = loc[1:2, :]
    dl2 = loc[2:3, :]
    dl3 = loc[3:4, :]
    dcx = pcx + dl0 * (VAR0) * pw
    dcy = pcy + dl1 * (VAR0) * ph
    dw = pw * jnp.exp(dl2 * VAR1)
    dh = ph * jnp.exp(dl3 * VAR1)
    dx1 = dcx - dw * 0.5
    dy1 = dcy - dh * 0.5
    dx2 = dx1 + dw
    dy2 = dy1 + dh
    c_iou = jaccard(dx1, dy1, dx2, dy2, dw * dh)
    c_iou = jnp.where(valid, c_iou, 0.0)

    cbps = jnp.max(c_iou, axis=0, keepdims=True)
    cbpi = jnp.min(jnp.where(c_iou == cbps, rowI, n_obj), axis=0, keepdims=True)

    iou_s[...] = c_iou * (c_iou >= T2).astype(f32)

    ords = []
    tscs = []
    for _ in range(K):
        cm = iou_s[...]
        tk = jnp.max(cm, axis=1, keepdims=True)
        ok = jnp.min(jnp.where(cm == tk, colI, p_pad), axis=1, keepdims=True)
        iou_s[...] = jnp.where(colI == ok, -1.0, cm)
        ords.append(ok)
        tscs.append(tk)

    cgs = [jnp.sum(jnp.where(colI == ok, conf1, 0.0), axis=1, keepdims=True)
           for ok in ords]

    cbpi_s[...] = cbpi
    cps_s[...] = jnp.zeros((1, p_pad), f32)

    def scatter_body(i, carry):
        for k in range(K):
            p = jnp.sum(jnp.where(rowV == i, ords[k], 0))
            ts = jnp.sum(jnp.where(rowV == i, tscs[k], 0.0))
            cg = jnp.sum(jnp.where(rowV == i, cgs[k], 0.0))
            hit = (cg < 1.0) & (ts > 0.0)
            m = (col1 == p) & hit
            cbpi_s[...] = jnp.where(m, i, cbpi_s[...])
            cps_s[...] = jnp.where(m, ts, cps_s[...])
        return carry

    jax.lax.fori_loop(0, n_obj, scatter_body, 0)

    cps = cps_s[...]
    g2 = gather_rows(cbpi_s[...])
    conf2 = jnp.where(cps < T2, -1.0, g2[4:5, :])
    enc2 = encode(g2)

    ign = (bts < T1) & (~(cbps < T2)) & (cps < T2)
    conf1 = jnp.where(ign, -1.0, conf1)

    validf = valid.astype(f32)
    m1 = ((conf1 > 0) & valid).astype(f32)
    m2 = ((conf2 > 0) & valid).astype(f32)
    n1 = jnp.sum(m1)
    n2 = jnp.sum(m2)

    def smooth_l1(enc, m):
        s = jnp.zeros((), f32)
        for r in range(4):
            x = jnp.abs(loc[r:r + 1, :] - enc[r])
            l = jnp.where(x >= BETA, x - 0.5 * BETA, 0.5 * x * x / BETA)
            s = s + jnp.sum(l * m)
        return s

    sl1 = smooth_l1(enc1, m1)
    sl2 = smooth_l1(enc2, m2)

    c0 = conf_ref[0][0:1, :]
    c1 = conf_ref[0][1:2, :]

    def focal(t_row, fiou):
        keep = ((t_row >= 0) & valid).astype(f32)
        t = jnp.maximum(t_row, 0.0)
        x = jnp.where(t >= 0.5, c1, c0)
        ce = jnp.maximum(x, 0.0) - x * t + jnp.log1p(jnp.exp(-jnp.abs(x)))
        a = t * ALPHA + (1.0 - t) * (1.0 - ALPHA)
        if fiou is not None:
            a = a * fiou
        sig = 1.0 / (1.0 + jnp.exp(-x))
        pt = jnp.where(t == 1.0, sig, 1.0 - sig)
        om = 1.0 - pt
        return jnp.sum(a * om * om * ce * keep)

    f1 = focal(conf1, None)
    f2 = focal(conf2, cps)

    lane = jax.lax.broadcasted_iota(i32, (1, 128), 1)
    outv = (jnp.where(lane == 0, n1, 0.0) + jnp.where(lane == 1, n2, 0.0)
            + jnp.where(lane == 2, sl1, 0.0) + jnp.where(lane == 3, sl2, 0.0)
            + jnp.where(lane == 4, f1, 0.0) + jnp.where(lane == 5, f2, 0.0))
    out_ref[...] = outv[None]


def kernel(loc_data, conf_data, priors, targets, im_names):
    B, P, _ = loc_data.shape
    n_obj = targets.shape[1]
    p_pad = ((P + 127) // 128) * 128

    locT = jnp.pad(jnp.transpose(loc_data, (0, 2, 1)),
                   ((0, 0), (0, 0), (0, p_pad - P)))
    confT = jnp.pad(jnp.transpose(conf_data, (0, 2, 1)),
                    ((0, 0), (0, 0), (0, p_pad - P)))
    priT = jnp.transpose(priors, (1, 0))
    pad_col = jnp.array([5.0, 5.0, 0.1, 0.1], jnp.float32)[:, None]
    priT = jnp.concatenate(
        [priT, jnp.broadcast_to(pad_col, (4, p_pad - P))], axis=1)
    tgtT = jnp.transpose(targets, (0, 2, 1))
    tx1 = targets[:, :, 0:1]
    ty1 = targets[:, :, 1:2]
    tx2 = targets[:, :, 2:3]
    ty2 = targets[:, :, 3:4]
    tlab = targets[:, :, 4:5]

    body = functools.partial(_hamloss_body, n_obj=n_obj, p_real=P,
                             p_pad=p_pad)
    out = pl.pallas_call(
        body,
        grid=(B,),
        in_specs=[
            pl.BlockSpec((1, 4, p_pad), lambda b: (b, 0, 0)),
            pl.BlockSpec((1, 2, p_pad), lambda b: (b, 0, 0)),
            pl.BlockSpec((4, p_pad), lambda b: (0, 0)),
            pl.BlockSpec((1, 5, n_obj), lambda b: (b, 0, 0)),
            pl.BlockSpec((1, n_obj, 1), lambda b: (b, 0, 0)),
            pl.BlockSpec((1, n_obj, 1), lambda b: (b, 0, 0)),
            pl.BlockSpec((1, n_obj, 1), lambda b: (b, 0, 0)),
            pl.BlockSpec((1, n_obj, 1), lambda b: (b, 0, 0)),
            pl.BlockSpec((1, n_obj, 1), lambda b: (b, 0, 0)),
        ],
        out_specs=pl.BlockSpec((1, 1, 128), lambda b: (b, 0, 0)),
        out_shape=jax.ShapeDtypeStruct((B, 1, 128), jnp.float32),
        scratch_shapes=[
            pltpu.VMEM((n_obj, p_pad), jnp.float32),
            pltpu.VMEM((1, p_pad), jnp.float32),
            pltpu.VMEM((1, p_pad), jnp.int32),
            pltpu.VMEM((1, p_pad), jnp.int32),
            pltpu.VMEM((1, p_pad), jnp.float32),
        ],
        compiler_params=pltpu.CompilerParams(
            dimension_semantics=("parallel",)),
    )(locT, confT, priT, tgtT, tx1, ty1, tx2, ty2, tlab)

    n1 = jnp.sum(out[:, 0, 0])
    n2 = jnp.sum(out[:, 0, 1])
    sl1 = jnp.sum(out[:, 0, 2])
    sl2 = jnp.sum(out[:, 0, 3])
    f1 = jnp.sum(out[:, 0, 4])
    f2 = jnp.sum(out[:, 0, 5])

    has1 = n1 > 0
    has2 = n2 > 0
    ll1 = sl1 / n1
    ll2 = sl2 / n2
    cl1 = f1 / n1
    cl2 = f2 / n2
    fallback = jnp.asarray(0.0001, jnp.float32)
    loc_loss = jnp.where(has1 & (~has2), ll1,
               jnp.where(has1 & has2, ll1 + ll2,
               jnp.where((~has1) & has2, ll2, fallback)))
    cls_loss = jnp.where(has1 & (~has2), cl1,
               jnp.where(has1 & has2, cl1 + cl2,
               jnp.where((~has1) & has2, cl2, fallback)))
    return (loc_loss, cls_loss)


# speedup vs baseline: 20.3100x; 1.0512x over previous
import functools

import jax
import jax.numpy as jnp
from jax.experimental import pallas as pl
from jax.experimental.pallas import tpu as pltpu

VAR0 = 0.1
VAR1 = 0.2
K = 5
T1 = 0.35
T2 = 0.5
ALPHA = 0.25
GAMMA = 2.0
BETA = 0.11


def _hamloss_body(loc_ref, conf_ref, pri_ref, tgtT_ref,
                  tx1_ref, ty1_ref, tx2_ref, ty2_ref, tlab_ref,
                  out_ref, iou_s, bts_s, bti_s, cbpi_s, cps_s,
                  elim_s, rdead_s, *, n_obj, p_real, p_pad):
    f32 = jnp.float32
    i32 = jnp.int32

    loc = loc_ref[0]
    pcx = pri_ref[0:1, :]
    pcy = pri_ref[1:2, :]
    pw = pri_ref[2:3, :]
    ph = pri_ref[3:4, :]
    tgtT = tgtT_ref[0]
    tx1 = tx1_ref[0]
    ty1 = ty1_ref[0]
    tx2 = tx2_ref[0]
    ty2 = ty2_ref[0]

    colI = jax.lax.broadcasted_iota(i32, (n_obj, p_pad), 1)
    rowI = jax.lax.broadcasted_iota(i32, (n_obj, p_pad), 0)
    col1 = jax.lax.broadcasted_iota(i32, (1, p_pad), 1)
    rowV = jax.lax.broadcasted_iota(i32, (n_obj, 1), 0)
    valid = col1 < p_real

    area_t = (tx2 - tx1) * (ty2 - ty1)

    def jaccard(bx1, by1, bx2, by2, area_b):
        ltx = jnp.maximum(tx1, bx1)
        lty = jnp.maximum(ty1, by1)
        rbx = jnp.minimum(tx2, bx2)
        rby = jnp.minimum(ty2, by2)
        iw = jnp.clip(rbx - ltx, 0.0, None)
        ih = jnp.clip(rby - lty, 0.0, None)
        inter = iw * ih
        return inter / (area_t + area_b - inter)

    px1 = pcx - pw * 0.5
    py1 = pcy - ph * 0.5
    px2 = pcx + pw * 0.5
    py2 = pcy + ph * 0.5
    iou0 = jaccard(px1, py1, px2, py2, pw * ph)
    iou0 = jnp.where(valid, iou0, -1.0)
    iou_s[...] = iou0

    bts0 = jnp.max(iou0, axis=0, keepdims=True)
    bti0 = jnp.min(jnp.where(iou0 == bts0, rowI, n_obj), axis=0, keepdims=True)
    bts_s[...] = bts0
    bti_s[...] = bti0

    elim_s[...] = (~valid).astype(i32)
    rdead_s[...] = jnp.zeros((n_obj, 1), i32)

    def greedy_body(_, carry):
        elim = elim_s[...] != 0
        row_dead = rdead_s[...] != 0
        iou = iou_s[...]
        bps = jnp.max(jnp.where(elim, -1.0, iou), axis=1, keepdims=True)
        bps = jnp.where(row_dead, -1.0, bps)
        val = jnp.max(bps)
        j = jnp.min(jnp.where(bps == val, rowV, n_obj))
        rowj = iou_s[pl.ds(j, 1), :]
        i = jnp.min(jnp.where((rowj == val) & (~elim), col1, p_pad))
        mi = col1 == i
        bts_s[...] = jnp.where(mi, val, bts_s[...])
        bti_s[...] = jnp.where(mi, j, bti_s[...])
        elim_s[...] = (elim | mi).astype(i32)
        rdead_s[...] = (row_dead | (rowV == j)).astype(i32)
        return carry

    jax.lax.fori_loop(0, n_obj, greedy_body, 0)

    bts = bts_s[...]
    bti = bti_s[...]

    def gather_rows(idx_row):
        oh = (idx_row == rowI).astype(f32)
        return jax.lax.dot_general(tgtT, oh, (((1,), (0,)), ((), ())),
                                   preferred_element_type=f32)

    g1 = gather_rows(bti)
    conf1 = jnp.where(bts < T1, 0.0, g1[4:5, :])

    def encode(g):
        mx1 = g[0:1, :]
        my1 = g[1:2, :]
        mx2 = g[2:3, :]
        my2 = g[3:4, :]
        e0 = ((mx1 + mx2) * 0.5 - pcx) / (VAR0 * pw)
        e1 = ((my1 + my2) * 0.5 - pcy) / (VAR0 * ph)
        e2 = jnp.log(jnp.clip((mx2 - mx1) / pw, 1e-8, None)) / VAR1
        e3 = jnp.log(jnp.clip((my2 - my1) / ph, 1e-8, None)) / VAR1
        return e0, e1, e2, e3

    enc1 = encode(g1)

    dl0 = loc[0:1, :]
    dl1 = loc[1:2, :]
    dl2 = loc[2:3, :]
    dl3 = loc[3:4, :]
    dcx = pcx + dl0 * (VAR0) * pw
    dcy = pcy + dl1 * (VAR0) * ph
    dw = pw * jnp.exp(dl2 * VAR1)
    dh = ph * jnp.exp(dl3 * VAR1)
    dx1 = dcx - dw * 0.5
    dy1 = dcy - dh * 0.5
    dx2 = dx1 + dw
    dy2 = dy1 + dh
    c_iou = jaccard(dx1, dy1, dx2, dy2, dw * dh)
    c_iou = jnp.where(valid, c_iou, 0.0)

    cbps = jnp.max(c_iou, axis=0, keepdims=True)
    cbpi = jnp.min(jnp.where(c_iou == cbps, rowI, n_obj), axis=0, keepdims=True)

    iou_s[...] = c_iou * (c_iou >= T2).astype(f32)

    ords = []
    tscs = []
    for _ in range(K):
        cm = iou_s[...]
        tk = jnp.max(cm, axis=1, keepdims=True)
        ok = jnp.min(jnp.where(cm == tk, colI, p_pad), axis=1, keepdims=True)
        iou_s[...] = jnp.where(colI == ok, -1.0, cm)
        ords.append(ok)
        tscs.append(tk)

    cgs = [jnp.sum(jnp.where(colI == ok, conf1, 0.0), axis=1, keepdims=True)
           for ok in ords]

    cbpi_s[...] = cbpi
    cps_s[...] = jnp.zeros((1, p_pad), f32)

    def scatter_body(i, carry):
        for k in range(K):
            p = jnp.sum(jnp.where(rowV == i, ords[k], 0))
            ts = jnp.sum(jnp.where(rowV == i, tscs[k], 0.0))
            cg = jnp.sum(jnp.where(rowV == i, cgs[k], 0.0))
            hit = (cg < 1.0) & (ts > 0.0)
            m = (col1 == p) & hit
            cbpi_s[...] = jnp.where(m, i, cbpi_s[...])
            cps_s[...] = jnp.where(m, ts, cps_s[...])
        return carry

    jax.lax.fori_loop(0, n_obj, scatter_body, 0)

    cps = cps_s[...]
    g2 = gather_rows(cbpi_s[...])
    conf2 = jnp.where(cps < T2, -1.0, g2[4:5, :])
    enc2 = encode(g2)

    ign = (bts < T1) & (~(cbps < T2)) & (cps < T2)
    conf1 = jnp.where(ign, -1.0, conf1)

    validf = valid.astype(f32)
    m1 = ((conf1 > 0) & valid).astype(f32)
    m2 = ((conf2 > 0) & valid).astype(f32)
    n1 = jnp.sum(m1)
    n2 = jnp.sum(m2)

    def smooth_l1(enc, m):
        s = jnp.zeros((), f32)
        for r in range(4):
            x = jnp.abs(loc[r:r + 1, :] - enc[r])
            l = jnp.where(x >= BETA, x - 0.5 * BETA, 0.5 * x * x / BETA)
            s = s + jnp.sum(l * m)
        return s

    sl1 = smooth_l1(enc1, m1)
    sl2 = smooth_l1(enc2, m2)

    c0 = conf_ref[0][0:1, :]
    c1 = conf_ref[0][1:2, :]

    def focal(t_row, fiou):
        keep = ((t_row >= 0) & valid).astype(f32)
        t = jnp.maximum(t_row, 0.0)
        x = jnp.where(t >= 0.5, c1, c0)
        ce = jnp.maximum(x, 0.0) - x * t + jnp.log1p(jnp.exp(-jnp.abs(x)))
        a = t * ALPHA + (1.0 - t) * (1.0 - ALPHA)
        if fiou is not None:
            a = a * fiou
        sig = 1.0 / (1.0 + jnp.exp(-x))
        pt = jnp.where(t == 1.0, sig, 1.0 - sig)
        om = 1.0 - pt
        return jnp.sum(a * om * om * ce * keep)

    f1 = focal(conf1, None)
    f2 = focal(conf2, cps)

    lane = jax.lax.broadcasted_iota(i32, (1, 128), 1)
    outv = (jnp.where(lane == 0, n1, 0.0) + jnp.where(lane == 1, n2, 0.0)
            + jnp.where(lane == 2, sl1, 0.0) + jnp.where(lane == 3, sl2, 0.0)
            + jnp.where(lane == 4, f1, 0.0) + jnp.where(lane == 5, f2, 0.0))
    out_ref[...] = outv[None]


def kernel(loc_data, conf_data, priors, targets, im_names):
    B, P, _ = loc_data.shape
    n_obj = targets.shape[1]
    p_pad = ((P + 127) // 128) * 128

    locT = jnp.pad(jnp.transpose(loc_data, (0, 2, 1)),
                   ((0, 0), (0, 0), (0, p_pad - P)))
    confT = jnp.pad(jnp.transpose(conf_data, (0, 2, 1)),
                    ((0, 0), (0, 0), (0, p_pad - P)))
    priT = jnp.transpose(priors, (1, 0))
    pad_col = jnp.array([5.0, 5.0, 0.1, 0.1], jnp.float32)[:, None]
    priT = jnp.concatenate(
        [priT, jnp.broadcast_to(pad_col, (4, p_pad - P))], axis=1)
    tgtT = jnp.transpose(targets, (0, 2, 1))
    tx1 = targets[:, :, 0:1]
    ty1 = targets[:, :, 1:2]
    tx2 = targets[:, :, 2:3]
    ty2 = targets[:, :, 3:4]
    tlab = targets[:, :, 4:5]

    body = functools.partial(_hamloss_body, n_obj=n_obj, p_real=P,
                             p_pad=p_pad)
    out = pl.pallas_call(
        body,
        grid=(B,),
        in_specs=[
            pl.BlockSpec((1, 4, p_pad), lambda b: (b, 0, 0)),
            pl.BlockSpec((1, 2, p_pad), lambda b: (b, 0, 0)),
            pl.BlockSpec((4, p_pad), lambda b: (0, 0)),
            pl.BlockSpec((1, 5, n_obj), lambda b: (b, 0, 0)),
            pl.BlockSpec((1, n_obj, 1), lambda b: (b, 0, 0)),
            pl.BlockSpec((1, n_obj, 1), lambda b: (b, 0, 0)),
            pl.BlockSpec((1, n_obj, 1), lambda b: (b, 0, 0)),
            pl.BlockSpec((1, n_obj, 1), lambda b: (b, 0, 0)),
            pl.BlockSpec((1, n_obj, 1), lambda b: (b, 0, 0)),
        ],
        out_specs=pl.BlockSpec((1, 1, 128), lambda b: (b, 0, 0)),
        out_shape=jax.ShapeDtypeStruct((B, 1, 128), jnp.float32),
        scratch_shapes=[
            pltpu.VMEM((n_obj, p_pad), jnp.float32),
            pltpu.VMEM((1, p_pad), jnp.float32),
            pltpu.VMEM((1, p_pad), jnp.int32),
            pltpu.VMEM((1, p_pad), jnp.int32),
            pltpu.VMEM((1, p_pad), jnp.float32),
            pltpu.VMEM((1, p_pad), jnp.int32),
            pltpu.VMEM((n_obj, 1), jnp.int32),
        ],
        compiler_params=pltpu.CompilerParams(
            dimension_semantics=("parallel",)),
    )(locT, confT, priT, tgtT, tx1, ty1, tx2, ty2, tlab)

    n1 = jnp.sum(out[:, 0, 0])
    n2 = jnp.sum(out[:, 0, 1])
    sl1 = jnp.sum(out[:, 0, 2])
    sl2 = jnp.sum(out[:, 0, 3])
    f1 = jnp.sum(out[:, 0, 4])
    f2 = jnp.sum(out[:, 0, 5])

    has1 = n1 > 0
    has2 = n2 > 0
    ll1 = sl1 / n1
    ll2 = sl2 / n2
    cl1 = f1 / n1
    cl2 = f2 / n2
    fallback = jnp.asarray(0.0001, jnp.float32)
    loc_loss = jnp.where(has1 & (~has2), ll1,
               jnp.where(has1 & has2, ll1 + ll2,
               jnp.where((~has1) & has2, ll2, fallback)))
    cls_loss = jnp.where(has1 & (~has2), cl1,
               jnp.where(has1 & has2, cl1 + cl2,
               jnp.where((~has1) & has2, cl2, fallback)))
    return (loc_loss, cls_loss)


# speedup vs baseline: 31.0408x; 1.5284x over previous
import functools

import jax
import jax.numpy as jnp
from jax.experimental import pallas as pl
from jax.experimental.pallas import tpu as pltpu

VAR0 = 0.1
VAR1 = 0.2
K = 5
T1 = 0.35
T2 = 0.5
ALPHA = 0.25
GAMMA = 2.0
BETA = 0.11


def _hamloss_body(loc_ref, conf_ref, pri_ref, tgtT_ref,
                  tx1_ref, ty1_ref, tx2_ref, ty2_ref, tlab_ref,
                  out_ref, iou_s, bts_s, bti_s,
                  elim_s, rdead_s, *, n_obj, p_real, p_pad):
    f32 = jnp.float32
    i32 = jnp.int32

    loc = loc_ref[0]
    pcx = pri_ref[0:1, :]
    pcy = pri_ref[1:2, :]
    pw = pri_ref[2:3, :]
    ph = pri_ref[3:4, :]
    tgtT = tgtT_ref[0]
    tx1 = tx1_ref[0]
    ty1 = ty1_ref[0]
    tx2 = tx2_ref[0]
    ty2 = ty2_ref[0]

    colI = jax.lax.broadcasted_iota(i32, (n_obj, p_pad), 1)
    rowI = jax.lax.broadcasted_iota(i32, (n_obj, p_pad), 0)
    col1 = jax.lax.broadcasted_iota(i32, (1, p_pad), 1)
    rowV = jax.lax.broadcasted_iota(i32, (n_obj, 1), 0)
    valid = col1 < p_real

    area_t = (tx2 - tx1) * (ty2 - ty1)

    def jaccard(bx1, by1, bx2, by2, area_b):
        ltx = jnp.maximum(tx1, bx1)
        lty = jnp.maximum(ty1, by1)
        rbx = jnp.minimum(tx2, bx2)
        rby = jnp.minimum(ty2, by2)
        iw = jnp.clip(rbx - ltx, 0.0, None)
        ih = jnp.clip(rby - lty, 0.0, None)
        inter = iw * ih
        return inter / (area_t + area_b - inter)

    px1 = pcx - pw * 0.5
    py1 = pcy - ph * 0.5
    px2 = pcx + pw * 0.5
    py2 = pcy + ph * 0.5
    iou0 = jaccard(px1, py1, px2, py2, pw * ph)
    iou0 = jnp.where(valid, iou0, -1.0)
    iou_s[...] = iou0

    bts0 = jnp.max(iou0, axis=0, keepdims=True)
    bti0 = jnp.min(jnp.where(iou0 == bts0, rowI, n_obj), axis=0, keepdims=True)
    bts_s[...] = bts0
    bti_s[...] = bti0

    elim_s[...] = (~valid).astype(i32)
    rdead_s[...] = jnp.zeros((n_obj, 1), i32)

    def greedy_body(_, carry):
        elim = elim_s[...] != 0
        row_dead = rdead_s[...] != 0
        iou = iou_s[...]
        bps = jnp.max(jnp.where(elim, -1.0, iou), axis=1, keepdims=True)
        bps = jnp.where(row_dead, -1.0, bps)
        val = jnp.max(bps)
        j = jnp.min(jnp.where(bps == val, rowV, n_obj))
        rowj = iou_s[pl.ds(j, 1), :]
        i = jnp.min(jnp.where((rowj == val) & (~elim), col1, p_pad))
        mi = col1 == i
        bts_s[...] = jnp.where(mi, val, bts_s[...])
        bti_s[...] = jnp.where(mi, j, bti_s[...])
        elim_s[...] = (elim | mi).astype(i32)
        rdead_s[...] = (row_dead | (rowV == j)).astype(i32)
        return carry

    jax.lax.fori_loop(0, n_obj, greedy_body, 0)

    bts = bts_s[...]
    bti = bti_s[...]

    def gather_rows(idx_row):
        oh = (idx_row == rowI).astype(f32)
        return jax.lax.dot_general(tgtT, oh, (((1,), (0,)), ((), ())),
                                   preferred_element_type=f32)

    g1 = gather_rows(bti)
    conf1 = jnp.where(bts < T1, 0.0, g1[4:5, :])

    def encode(g):
        mx1 = g[0:1, :]
        my1 = g[1:2, :]
        mx2 = g[2:3, :]
        my2 = g[3:4, :]
        e0 = ((mx1 + mx2) * 0.5 - pcx) / (VAR0 * pw)
        e1 = ((my1 + my2) * 0.5 - pcy) / (VAR0 * ph)
        e2 = jnp.log(jnp.clip((mx2 - mx1) / pw, 1e-8, None)) / VAR1
        e3 = jnp.log(jnp.clip((my2 - my1) / ph, 1e-8, None)) / VAR1
        return e0, e1, e2, e3

    enc1 = encode(g1)

    dl0 = loc[0:1, :]
    dl1 = loc[1:2, :]
    dl2 = loc[2:3, :]
    dl3 = loc[3:4, :]
    dcx = pcx + dl0 * (VAR0) * pw
    dcy = pcy + dl1 * (VAR0) * ph
    dw = pw * jnp.exp(dl2 * VAR1)
    dh = ph * jnp.exp(dl3 * VAR1)
    dx1 = dcx - dw * 0.5
    dy1 = dcy - dh * 0.5
    dx2 = dx1 + dw
    dy2 = dy1 + dh
    c_iou = jaccard(dx1, dy1, dx2, dy2, dw * dh)
    c_iou = jnp.where(valid, c_iou, 0.0)

    cbps = jnp.max(c_iou, axis=0, keepdims=True)
    cbpi = jnp.min(jnp.where(c_iou == cbps, rowI, n_obj), axis=0, keepdims=True)

    iou_s[...] = c_iou * (c_iou >= T2).astype(f32)

    ords = []
    tscs = []
    for _ in range(K):
        cm = iou_s[...]
        tk = jnp.max(cm, axis=1, keepdims=True)
        ok = jnp.min(jnp.where(cm == tk, colI, p_pad), axis=1, keepdims=True)
        iou_s[...] = jnp.where(colI == ok, -1.0, cm)
        ords.append(ok)
        tscs.append(tk)

    masks = [colI == ok for ok in ords]
    cgs = [jnp.sum(jnp.where(m, conf1, 0.0), axis=1, keepdims=True)
           for m in masks]
    best = None
    for k in range(K):
        hit_k = (cgs[k] < 1.0) & (tscs[k] > 0.0)
        prio_k = jnp.where(hit_k, rowV * K + k, -1)
        c = jnp.where(masks[k], prio_k, -1)
        best = c if best is None else jnp.maximum(best, c)
    bestprio = jnp.max(best, axis=0, keepdims=True)
    tsel = None
    for k in range(K):
        c = jnp.where(masks[k] & ((rowV * K + k) == bestprio), tscs[k], 0.0)
        tsel = c if tsel is None else jnp.maximum(tsel, c)
    ts_win = jnp.max(tsel, axis=0, keepdims=True)
    hitcol = bestprio >= 0
    cps = jnp.where(hitcol, ts_win, 0.0)
    cbpi_f = jnp.where(hitcol, bestprio // K, cbpi)
    g2 = gather_rows(cbpi_f)
    conf2 = jnp.where(cps < T2, -1.0, g2[4:5, :])
    enc2 = encode(g2)

    ign = (bts < T1) & (~(cbps < T2)) & (cps < T2)
    conf1 = jnp.where(ign, -1.0, conf1)

    validf = valid.astype(f32)
    m1 = ((conf1 > 0) & valid).astype(f32)
    m2 = ((conf2 > 0) & valid).astype(f32)
    n1 = jnp.sum(m1)
    n2 = jnp.sum(m2)

    def smooth_l1(enc, m):
        s = jnp.zeros((), f32)
        for r in range(4):
            x = jnp.abs(loc[r:r + 1, :] - enc[r])
            l = jnp.where(x >= BETA, x - 0.5 * BETA, 0.5 * x * x / BETA)
            s = s + jnp.sum(l * m)
        return s

    sl1 = smooth_l1(enc1, m1)
    sl2 = smooth_l1(enc2, m2)

    c0 = conf_ref[0][0:1, :]
    c1 = conf_ref[0][1:2, :]

    def focal(t_row, fiou):
        keep = ((t_row >= 0) & valid).astype(f32)
        t = jnp.maximum(t_row, 0.0)
        x = jnp.where(t >= 0.5, c1, c0)
        ce = jnp.maximum(x, 0.0) - x * t + jnp.log1p(jnp.exp(-jnp.abs(x)))
        a = t * ALPHA + (1.0 - t) * (1.0 - ALPHA)
        if fiou is not None:
            a = a * fiou
        sig = 1.0 / (1.0 + jnp.exp(-x))
        pt = jnp.where(t == 1.0, sig, 1.0 - sig)
        om = 1.0 - pt
        return jnp.sum(a * om * om * ce * keep)

    f1 = focal(conf1, None)
    f2 = focal(conf2, cps)

    lane = jax.lax.broadcasted_iota(i32, (1, 128), 1)
    outv = (jnp.where(lane == 0, n1, 0.0) + jnp.where(lane == 1, n2, 0.0)
            + jnp.where(lane == 2, sl1, 0.0) + jnp.where(lane == 3, sl2, 0.0)
            + jnp.where(lane == 4, f1, 0.0) + jnp.where(lane == 5, f2, 0.0))
    out_ref[...] = outv[None]


def kernel(loc_data, conf_data, priors, targets, im_names):
    B, P, _ = loc_data.shape
    n_obj = targets.shape[1]
    p_pad = ((P + 127) // 128) * 128

    locT = jnp.pad(jnp.transpose(loc_data, (0, 2, 1)),
                   ((0, 0), (0, 0), (0, p_pad - P)))
    confT = jnp.pad(jnp.transpose(conf_data, (0, 2, 1)),
                    ((0, 0), (0, 0), (0, p_pad - P)))
    priT = jnp.transpose(priors, (1, 0))
    pad_col = jnp.array([5.0, 5.0, 0.1, 0.1], jnp.float32)[:, None]
    priT = jnp.concatenate(
        [priT, jnp.broadcast_to(pad_col, (4, p_pad - P))], axis=1)
    tgtT = jnp.transpose(targets, (0, 2, 1))
    tx1 = targets[:, :, 0:1]
    ty1 = targets[:, :, 1:2]
    tx2 = targets[:, :, 2:3]
    ty2 = targets[:, :, 3:4]
    tlab = targets[:, :, 4:5]

    body = functools.partial(_hamloss_body, n_obj=n_obj, p_real=P,
                             p_pad=p_pad)
    out = pl.pallas_call(
        body,
        grid=(B,),
        in_specs=[
            pl.BlockSpec((1, 4, p_pad), lambda b: (b, 0, 0)),
            pl.BlockSpec((1, 2, p_pad), lambda b: (b, 0, 0)),
            pl.BlockSpec((4, p_pad), lambda b: (0, 0)),
            pl.BlockSpec((1, 5, n_obj), lambda b: (b, 0, 0)),
            pl.BlockSpec((1, n_obj, 1), lambda b: (b, 0, 0)),
            pl.BlockSpec((1, n_obj, 1), lambda b: (b, 0, 0)),
            pl.BlockSpec((1, n_obj, 1), lambda b: (b, 0, 0)),
            pl.BlockSpec((1, n_obj, 1), lambda b: (b, 0, 0)),
            pl.BlockSpec((1, n_obj, 1), lambda b: (b, 0, 0)),
        ],
        out_specs=pl.BlockSpec((1, 1, 128), lambda b: (b, 0, 0)),
        out_shape=jax.ShapeDtypeStruct((B, 1, 128), jnp.float32),
        scratch_shapes=[
            pltpu.VMEM((n_obj, p_pad), jnp.float32),
            pltpu.VMEM((1, p_pad), jnp.float32),
            pltpu.VMEM((1, p_pad), jnp.int32),
            pltpu.VMEM((1, p_pad), jnp.int32),
            pltpu.VMEM((n_obj, 1), jnp.int32),
        ],
        compiler_params=pltpu.CompilerParams(
            dimension_semantics=("parallel",)),
    )(locT, confT, priT, tgtT, tx1, ty1, tx2, ty2, tlab)

    n1 = jnp.sum(out[:, 0, 0])
    n2 = jnp.sum(out[:, 0, 1])
    sl1 = jnp.sum(out[:, 0, 2])
    sl2 = jnp.sum(out[:, 0, 3])
    f1 = jnp.sum(out[:, 0, 4])
    f2 = jnp.sum(out[:, 0, 5])

    has1 = n1 > 0
    has2 = n2 > 0
    ll1 = sl1 / n1
    ll2 = sl2 / n2
    cl1 = f1 / n1
    cl2 = f2 / n2
    fallback = jnp.asarray(0.0001, jnp.float32)
    loc_loss = jnp.where(has1 & (~has2), ll1,
               jnp.where(has1 & has2, ll1 + ll2,
               jnp.where((~has1) & has2, ll2, fallback)))
    cls_loss = jnp.where(has1 & (~has2), cl1,
               jnp.where(has1 & has2, cl1 + cl2,
               jnp.where((~has1) & has2, cl2, fallback)))
    return (loc_loss, cls_loss)


# speedup vs baseline: 38.5031x; 1.2404x over previous
import functools

import jax
import jax.numpy as jnp
from jax.experimental import pallas as pl
from jax.experimental.pallas import tpu as pltpu

VAR0 = 0.1
VAR1 = 0.2
K = 5
T1 = 0.35
T2 = 0.5
ALPHA = 0.25
GAMMA = 2.0
BETA = 0.11


def _hamloss_body(loc_ref, conf_ref, pri_ref, tgtT_ref,
                  tx1_ref, ty1_ref, tx2_ref, ty2_ref, tlab_ref,
                  out_ref, iou_s, elim_s, *, n_obj, p_real, p_pad):
    f32 = jnp.float32
    i32 = jnp.int32

    loc = loc_ref[0]
    pcx = pri_ref[0:1, :]
    pcy = pri_ref[1:2, :]
    pw = pri_ref[2:3, :]
    ph = pri_ref[3:4, :]
    tgtT = tgtT_ref[0]
    tx1 = tx1_ref[0]
    ty1 = ty1_ref[0]
    tx2 = tx2_ref[0]
    ty2 = ty2_ref[0]

    colI = jax.lax.broadcasted_iota(i32, (n_obj, p_pad), 1)
    rowI = jax.lax.broadcasted_iota(i32, (n_obj, p_pad), 0)
    col1 = jax.lax.broadcasted_iota(i32, (1, p_pad), 1)
    rowV = jax.lax.broadcasted_iota(i32, (n_obj, 1), 0)
    valid = col1 < p_real

    area_t = (tx2 - tx1) * (ty2 - ty1)

    def jaccard(bx1, by1, bx2, by2, area_b):
        ltx = jnp.maximum(tx1, bx1)
        lty = jnp.maximum(ty1, by1)
        rbx = jnp.minimum(tx2, bx2)
        rby = jnp.minimum(ty2, by2)
        iw = jnp.clip(rbx - ltx, 0.0, None)
        ih = jnp.clip(rby - lty, 0.0, None)
        inter = iw * ih
        return inter / (area_t + area_b - inter)

    px1 = pcx - pw * 0.5
    py1 = pcy - ph * 0.5
    px2 = pcx + pw * 0.5
    py2 = pcy + ph * 0.5
    iou0 = jaccard(px1, py1, px2, py2, pw * ph)
    iou0 = jnp.where(valid, iou0, -1.0)
    iou_s[...] = iou0

    bts0 = jnp.max(iou0, axis=0, keepdims=True)
    bti0 = jnp.min(jnp.where(iou0 == bts0, rowI, n_obj), axis=0, keepdims=True)

    elim_s[...] = (~valid).astype(i32)

    bps0 = jnp.max(iou0, axis=1, keepdims=True)
    bpi0 = jnp.min(jnp.where(iou0 == bps0, colI, p_pad), axis=1, keepdims=True)

    def greedy_body(t, carry):
        bps, bpi, rdead, js, iss, vals = carry

        def fix_cond(c):
            return ~c[-1]

        def fix_body(c):
            bps_c, bpi_c, _, _, _, _ = c
            bps_eff = jnp.where(rdead != 0, -1.0, bps_c)
            val = jnp.max(bps_eff)
            j = jnp.min(jnp.where(bps_eff == val, rowV, n_obj))
            i = jnp.sum(jnp.where(rowV == j, bpi_c, 0))
            elim = elim_s[...]
            stale = jnp.sum(jnp.where(col1 == i, elim, 0)) > 0
            rowj = iou_s[pl.ds(j, 1), :]
            rowm = jnp.where(elim != 0, -1.0, rowj)
            nv = jnp.max(rowm)
            ni = jnp.min(jnp.where(rowm == nv, col1, p_pad))
            upd = (rowV == j) & stale
            bps_c = jnp.where(upd, nv, bps_c)
            bpi_c = jnp.where(upd, ni, bpi_c)
            return (bps_c, bpi_c, j, i, val, ~stale)

        bps, bpi, j, i, val, _ = jax.lax.while_loop(
            fix_cond, fix_body,
            (bps, bpi, jnp.int32(0), jnp.int32(0), jnp.float32(0.0), False))

        elim_s[...] = elim_s[...] | (col1 == i).astype(i32)
        rdead = rdead | (rowV == j).astype(i32)
        upd = rowV == t
        js = jnp.where(upd, j, js)
        iss = jnp.where(upd, i, iss)
        vals = jnp.where(upd, val, vals)
        return (bps, bpi, rdead, js, iss, vals)

    zi = jnp.zeros((n_obj, 1), i32)
    _, _, _, js, iss, vals = jax.lax.fori_loop(
        0, n_obj, greedy_body,
        (bps0, bpi0, zi, zi, zi, jnp.zeros((n_obj, 1), jnp.float32)))

    ohp = col1 == iss
    bts_u = jnp.max(jnp.where(ohp, vals, -1e30), axis=0, keepdims=True)
    bti_u = jnp.min(jnp.where(ohp, js, n_obj + 1), axis=0, keepdims=True)
    picked = bts_u > -1e29
    bts = jnp.where(picked, bts_u, bts0)
    bti = jnp.where(picked, bti_u, bti0)

    def gather_rows(idx_row):
        oh = (idx_row == rowI).astype(f32)
        return jax.lax.dot_general(tgtT, oh, (((1,), (0,)), ((), ())),
                                   preferred_element_type=f32)

    g1 = gather_rows(bti)
    conf1 = jnp.where(bts < T1, 0.0, g1[4:5, :])

    def encode(g):
        mx1 = g[0:1, :]
        my1 = g[1:2, :]
        mx2 = g[2:3, :]
        my2 = g[3:4, :]
        e0 = ((mx1 + mx2) * 0.5 - pcx) / (VAR0 * pw)
        e1 = ((my1 + my2) * 0.5 - pcy) / (VAR0 * ph)
        e2 = jnp.log(jnp.clip((mx2 - mx1) / pw, 1e-8, None)) / VAR1
        e3 = jnp.log(jnp.clip((my2 - my1) / ph, 1e-8, None)) / VAR1
        return e0, e1, e2, e3

    enc1 = encode(g1)

    dl0 = loc[0:1, :]
    dl1 = loc[1:2, :]
    dl2 = loc[2:3, :]
    dl3 = loc[3:4, :]
    dcx = pcx + dl0 * (VAR0) * pw
    dcy = pcy + dl1 * (VAR0) * ph
    dw = pw * jnp.exp(dl2 * VAR1)
    dh = ph * jnp.exp(dl3 * VAR1)
    dx1 = dcx - dw * 0.5
    dy1 = dcy - dh * 0.5
    dx2 = dx1 + dw
    dy2 = dy1 + dh
    c_iou = jaccard(dx1, dy1, dx2, dy2, dw * dh)
    c_iou = jnp.where(valid, c_iou, 0.0)

    cbps = jnp.max(c_iou, axis=0, keepdims=True)
    cbpi = jnp.min(jnp.where(c_iou == cbps, rowI, n_obj), axis=0, keepdims=True)

    iou_s[...] = c_iou * (c_iou >= T2).astype(f32)

    ords = []
    tscs = []
    for _ in range(K):
        cm = iou_s[...]
        tk = jnp.max(cm, axis=1, keepdims=True)
        ok = jnp.min(jnp.where(cm == tk, colI, p_pad), axis=1, keepdims=True)
        iou_s[...] = jnp.where(colI == ok, -1.0, cm)
        ords.append(ok)
        tscs.append(tk)

    masks = [colI == ok for ok in ords]
    cgs = [jnp.sum(jnp.where(m, conf1, 0.0), axis=1, keepdims=True)
           for m in masks]
    best = None
    for k in range(K):
        hit_k = (cgs[k] < 1.0) & (tscs[k] > 0.0)
        prio_k = jnp.where(hit_k, rowV * K + k, -1)
        c = jnp.where(masks[k], prio_k, -1)
        best = c if best is None else jnp.maximum(best, c)
    bestprio = jnp.max(best, axis=0, keepdims=True)
    tsel = None
    for k in range(K):
        c = jnp.where(masks[k] & ((rowV * K + k) == bestprio), tscs[k], 0.0)
        tsel = c if tsel is None else jnp.maximum(tsel, c)
    ts_win = jnp.max(tsel, axis=0, keepdims=True)
    hitcol = bestprio >= 0
    cps = jnp.where(hitcol, ts_win, 0.0)
    cbpi_f = jnp.where(hitcol, bestprio // K, cbpi)
    g2 = gather_rows(cbpi_f)
    conf2 = jnp.where(cps < T2, -1.0, g2[4:5, :])
    enc2 = encode(g2)

    ign = (bts < T1) & (~(cbps < T2)) & (cps < T2)
    conf1 = jnp.where(ign, -1.0, conf1)

    validf = valid.astype(f32)
    m1 = ((conf1 > 0) & valid).astype(f32)
    m2 = ((conf2 > 0) & valid).astype(f32)
    n1 = jnp.sum(m1)
    n2 = jnp.sum(m2)

    def smooth_l1(enc, m):
        s = jnp.zeros((), f32)
        for r in range(4):
            x = jnp.abs(loc[r:r + 1, :] - enc[r])
            l = jnp.where(x >= BETA, x - 0.5 * BETA, 0.5 * x * x / BETA)
            s = s + jnp.sum(l * m)
        return s

    sl1 = smooth_l1(enc1, m1)
    sl2 = smooth_l1(enc2, m2)

    c0 = conf_ref[0][0:1, :]
    c1 = conf_ref[0][1:2, :]

    def focal(t_row, fiou):
        keep = ((t_row >= 0) & valid).astype(f32)
        t = jnp.maximum(t_row, 0.0)
        x = jnp.where(t >= 0.5, c1, c0)
        ce = jnp.maximum(x, 0.0) - x * t + jnp.log1p(jnp.exp(-jnp.abs(x)))
        a = t * ALPHA + (1.0 - t) * (1.0 - ALPHA)
        if fiou is not None:
            a = a * fiou
        sig = 1.0 / (1.0 + jnp.exp(-x))
        pt = jnp.where(t == 1.0, sig, 1.0 - sig)
        om = 1.0 - pt
        return jnp.sum(a * om * om * ce * keep)

    f1 = focal(conf1, None)
    f2 = focal(conf2, cps)

    lane = jax.lax.broadcasted_iota(i32, (1, 128), 1)
    outv = (jnp.where(lane == 0, n1, 0.0) + jnp.where(lane == 1, n2, 0.0)
            + jnp.where(lane == 2, sl1, 0.0) + jnp.where(lane == 3, sl2, 0.0)
            + jnp.where(lane == 4, f1, 0.0) + jnp.where(lane == 5, f2, 0.0))
    out_ref[...] = outv[None]


def kernel(loc_data, conf_data, priors, targets, im_names):
    B, P, _ = loc_data.shape
    n_obj = targets.shape[1]
    p_pad = ((P + 127) // 128) * 128

    locT = jnp.pad(jnp.transpose(loc_data, (0, 2, 1)),
                   ((0, 0), (0, 0), (0, p_pad - P)))
    confT = jnp.pad(jnp.transpose(conf_data, (0, 2, 1)),
                    ((0, 0), (0, 0), (0, p_pad - P)))
    priT = jnp.transpose(priors, (1, 0))
    pad_col = jnp.array([5.0, 5.0, 0.1, 0.1], jnp.float32)[:, None]
    priT = jnp.concatenate(
        [priT, jnp.broadcast_to(pad_col, (4, p_pad - P))], axis=1)
    tgtT = jnp.transpose(targets, (0, 2, 1))
    tx1 = targets[:, :, 0:1]
    ty1 = targets[:, :, 1:2]
    tx2 = targets[:, :, 2:3]
    ty2 = targets[:, :, 3:4]
    tlab = targets[:, :, 4:5]

    body = functools.partial(_hamloss_body, n_obj=n_obj, p_real=P,
                             p_pad=p_pad)
    out = pl.pallas_call(
        body,
        grid=(B,),
        in_specs=[
            pl.BlockSpec((1, 4, p_pad), lambda b: (b, 0, 0)),
            pl.BlockSpec((1, 2, p_pad), lambda b: (b, 0, 0)),
            pl.BlockSpec((4, p_pad), lambda b: (0, 0)),
            pl.BlockSpec((1, 5, n_obj), lambda b: (b, 0, 0)),
            pl.BlockSpec((1, n_obj, 1), lambda b: (b, 0, 0)),
            pl.BlockSpec((1, n_obj, 1), lambda b: (b, 0, 0)),
            pl.BlockSpec((1, n_obj, 1), lambda b: (b, 0, 0)),
            pl.BlockSpec((1, n_obj, 1), lambda b: (b, 0, 0)),
            pl.BlockSpec((1, n_obj, 1), lambda b: (b, 0, 0)),
        ],
        out_specs=pl.BlockSpec((1, 1, 128), lambda b: (b, 0, 0)),
        out_shape=jax.ShapeDtypeStruct((B, 1, 128), jnp.float32),
        scratch_shapes=[
            pltpu.VMEM((n_obj, p_pad), jnp.float32),
            pltpu.VMEM((1, p_pad), jnp.int32),
        ],
        compiler_params=pltpu.CompilerParams(
            dimension_semantics=("parallel",)),
    )(locT, confT, priT, tgtT, tx1, ty1, tx2, ty2, tlab)

    n1 = jnp.sum(out[:, 0, 0])
    n2 = jnp.sum(out[:, 0, 1])
    sl1 = jnp.sum(out[:, 0, 2])
    sl2 = jnp.sum(out[:, 0, 3])
    f1 = jnp.sum(out[:, 0, 4])
    f2 = jnp.sum(out[:, 0, 5])

    has1 = n1 > 0
    has2 = n2 > 0
    ll1 = sl1 / n1
    ll2 = sl2 / n2
    cl1 = f1 / n1
    cl2 = f2 / n2
    fallback = jnp.asarray(0.0001, jnp.float32)
    loc_loss = jnp.where(has1 & (~has2), ll1,
               jnp.where(has1 & has2, ll1 + ll2,
               jnp.where((~has1) & has2, ll2, fallback)))
    cls_loss = jnp.where(has1 & (~has2), cl1,
               jnp.where(has1 & has2, cl1 + cl2,
               jnp.where((~has1) & has2, cl2, fallback)))
    return (loc_loss, cls_loss)


# speedup vs baseline: 49.3145x; 1.2808x over previous
import functools

import jax
import jax.numpy as jnp
from jax.experimental import pallas as pl
from jax.experimental.pallas import tpu as pltpu

VAR0 = 0.1
VAR1 = 0.2
K = 5
T1 = 0.35
T2 = 0.5
ALPHA = 0.25
GAMMA = 2.0
BETA = 0.11


def _hamloss_body(loc_ref, conf_ref, pri_ref, tgtT_ref,
                  tx1_ref, ty1_ref, tx2_ref, ty2_ref, tlab_ref,
                  out_ref, iou_s, elim_s, *, n_obj, p_real, p_pad):
    f32 = jnp.float32
    i32 = jnp.int32

    loc = loc_ref[0]
    pcx = pri_ref[0:1, :]
    pcy = pri_ref[1:2, :]
    pw = pri_ref[2:3, :]
    ph = pri_ref[3:4, :]
    tgtT = tgtT_ref[0]
    tx1 = tx1_ref[0]
    ty1 = ty1_ref[0]
    tx2 = tx2_ref[0]
    ty2 = ty2_ref[0]

    colI = jax.lax.broadcasted_iota(i32, (n_obj, p_pad), 1)
    rowI = jax.lax.broadcasted_iota(i32, (n_obj, p_pad), 0)
    col1 = jax.lax.broadcasted_iota(i32, (1, p_pad), 1)
    rowV = jax.lax.broadcasted_iota(i32, (n_obj, 1), 0)
    valid = col1 < p_real

    area_t = (tx2 - tx1) * (ty2 - ty1)

    def jaccard(bx1, by1, bx2, by2, area_b):
        ltx = jnp.maximum(tx1, bx1)
        lty = jnp.maximum(ty1, by1)
        rbx = jnp.minimum(tx2, bx2)
        rby = jnp.minimum(ty2, by2)
        iw = jnp.clip(rbx - ltx, 0.0, None)
        ih = jnp.clip(rby - lty, 0.0, None)
        inter = iw * ih
        return inter / (area_t + area_b - inter)

    px1 = pcx - pw * 0.5
    py1 = pcy - ph * 0.5
    px2 = pcx + pw * 0.5
    py2 = pcy + ph * 0.5
    iou0 = jaccard(px1, py1, px2, py2, pw * ph)
    iou0 = jnp.where(valid, iou0, -1.0)
    iou_s[...] = iou0

    bts0 = jnp.max(iou0, axis=0, keepdims=True)
    bti0 = jnp.min(jnp.where(iou0 == bts0, rowI, n_obj), axis=0, keepdims=True)

    elim_s[...] = (~valid).astype(i32)

    bps0 = jnp.max(iou0, axis=1, keepdims=True)
    bpi0 = jnp.min(jnp.where(iou0 == bps0, colI, p_pad), axis=1, keepdims=True)

    def argpick(bps_c, bpi_c, rdead, elim):
        bps_eff = jnp.where(rdead != 0, -1.0, bps_c)
        val = jnp.max(bps_eff)
        j = jnp.min(jnp.where(bps_eff == val, rowV, n_obj))
        i = jnp.sum(jnp.where(rowV == j, bpi_c, 0))
        stale = jnp.sum(jnp.where(col1 == i, elim, 0)) > 0
        return j, i, val, stale

    def greedy_body(t, carry):
        bps, bpi, rdead, js, iss, vals = carry
        j0, i0, val0, stale0 = argpick(bps, bpi, rdead, elim_s[...])

        def fix_cond(c):
            return ~c[-1]

        def fix_body(c):
            bps_c, bpi_c, j, _, _, _ = c
            elim = elim_s[...]
            rowj = iou_s[pl.ds(j, 1), :]
            rowm = jnp.where(elim != 0, -1.0, rowj)
            nv = jnp.max(rowm)
            ni = jnp.min(jnp.where(rowm == nv, col1, p_pad))
            upd = rowV == j
            bps_c = jnp.where(upd, nv, bps_c)
            bpi_c = jnp.where(upd, ni, bpi_c)
            j2, i2, val2, stale2 = argpick(bps_c, bpi_c, rdead, elim)
            return (bps_c, bpi_c, j2, i2, val2, ~stale2)

        bps, bpi, j, i, val, _ = jax.lax.while_loop(
            fix_cond, fix_body, (bps, bpi, j0, i0, val0, ~stale0))

        elim_s[...] = elim_s[...] | (col1 == i).astype(i32)
        rdead = rdead | (rowV == j).astype(i32)
        upd = rowV == t
        js = jnp.where(upd, j, js)
        iss = jnp.where(upd, i, iss)
        vals = jnp.where(upd, val, vals)
        return (bps, bpi, rdead, js, iss, vals)

    zi = jnp.zeros((n_obj, 1), i32)
    _, _, _, js, iss, vals = jax.lax.fori_loop(
        0, n_obj, greedy_body,
        (bps0, bpi0, zi, zi, zi, jnp.zeros((n_obj, 1), jnp.float32)))

    ohp = col1 == iss
    bts_u = jnp.max(jnp.where(ohp, vals, -1e30), axis=0, keepdims=True)
    bti_u = jnp.min(jnp.where(ohp, js, n_obj + 1), axis=0, keepdims=True)
    picked = bts_u > -1e29
    bts = jnp.where(picked, bts_u, bts0)
    bti = jnp.where(picked, bti_u, bti0)

    def gather_rows(idx_row):
        oh = (idx_row == rowI).astype(f32)
        return jax.lax.dot_general(tgtT, oh, (((1,), (0,)), ((), ())),
                                   preferred_element_type=f32)

    g1 = gather_rows(bti)
    conf1 = jnp.where(bts < T1, 0.0, g1[4:5, :])

    def encode(g):
        mx1 = g[0:1, :]
        my1 = g[1:2, :]
        mx2 = g[2:3, :]
        my2 = g[3:4, :]
        e0 = ((mx1 + mx2) * 0.5 - pcx) / (VAR0 * pw)
        e1 = ((my1 + my2) * 0.5 - pcy) / (VAR0 * ph)
        e2 = jnp.log(jnp.clip((mx2 - mx1) / pw, 1e-8, None)) / VAR1
        e3 = jnp.log(jnp.clip((my2 - my1) / ph, 1e-8, None)) / VAR1
        return e0, e1, e2, e3

    enc1 = encode(g1)

    dl0 = loc[0:1, :]
    dl1 = loc[1:2, :]
    dl2 = loc[2:3, :]
    dl3 = loc[3:4, :]
    dcx = pcx + dl0 * (VAR0) * pw
    dcy = pcy + dl1 * (VAR0) * ph
    dw = pw * jnp.exp(dl2 * VAR1)
    dh = ph * jnp.exp(dl3 * VAR1)
    dx1 = dcx - dw * 0.5
    dy1 = dcy - dh * 0.5
    dx2 = dx1 + dw
    dy2 = dy1 + dh
    c_iou = jaccard(dx1, dy1, dx2, dy2, dw * dh)
    c_iou = jnp.where(valid, c_iou, 0.0)

    cbps = jnp.max(c_iou, axis=0, keepdims=True)
    cbpi = jnp.min(jnp.where(c_iou == cbps, rowI, n_obj), axis=0, keepdims=True)

    iou_s[...] = c_iou * (c_iou >= T2).astype(f32)

    ords = []
    tscs = []
    cgs = []
    for _ in range(K):
        cm = iou_s[...]
        tk = jnp.max(cm, axis=1, keepdims=True)
        ok = jnp.min(jnp.where(cm == tk, colI, p_pad), axis=1, keepdims=True)
        mk = colI == ok
        iou_s[...] = jnp.where(mk, -1.0, cm)
        cgs.append(jnp.sum(jnp.where(mk, conf1, 0.0), axis=1, keepdims=True))
        ords.append(ok)
        tscs.append(tk)

    masks = [colI == ok for ok in ords]
    best = None
    for k in range(K):
        hit_k = (cgs[k] < 1.0) & (tscs[k] > 0.0)
        prio_k = jnp.where(hit_k, rowV * K + k, -1)
        c = jnp.where(masks[k], prio_k, -1)
        best = c if best is None else jnp.maximum(best, c)
    bestprio = jnp.max(best, axis=0, keepdims=True)
    tsel = None
    for k in range(K):
        c = jnp.where(masks[k] & ((rowV * K + k) == bestprio), tscs[k], 0.0)
        tsel = c if tsel is None else jnp.maximum(tsel, c)
    ts_win = jnp.max(tsel, axis=0, keepdims=True)
    hitcol = bestprio >= 0
    cps = jnp.where(hitcol, ts_win, 0.0)
    cbpi_f = jnp.where(hitcol, bestprio // K, cbpi)
    g2 = gather_rows(cbpi_f)
    conf2 = jnp.where(cps < T2, -1.0, g2[4:5, :])
    enc2 = encode(g2)

    ign = (bts < T1) & (~(cbps < T2)) & (cps < T2)
    conf1 = jnp.where(ign, -1.0, conf1)

    validf = valid.astype(f32)
    m1 = ((conf1 > 0) & valid).astype(f32)
    m2 = ((conf2 > 0) & valid).astype(f32)
    n1 = jnp.sum(m1)
    n2 = jnp.sum(m2)

    def smooth_l1(enc, m):
        s = jnp.zeros((), f32)
        for r in range(4):
            x = jnp.abs(loc[r:r + 1, :] - enc[r])
            l = jnp.where(x >= BETA, x - 0.5 * BETA, 0.5 * x * x / BETA)
            s = s + jnp.sum(l * m)
        return s

    sl1 = smooth_l1(enc1, m1)
    sl2 = smooth_l1(enc2, m2)

    c0 = conf_ref[0][0:1, :]
    c1 = conf_ref[0][1:2, :]

    def focal(t_row, fiou):
        keep = ((t_row >= 0) & valid).astype(f32)
        t = jnp.maximum(t_row, 0.0)
        x = jnp.where(t >= 0.5, c1, c0)
        ce = jnp.maximum(x, 0.0) - x * t + jnp.log1p(jnp.exp(-jnp.abs(x)))
        a = t * ALPHA + (1.0 - t) * (1.0 - ALPHA)
        if fiou is not None:
            a = a * fiou
        sig = 1.0 / (1.0 + jnp.exp(-x))
        pt = jnp.where(t == 1.0, sig, 1.0 - sig)
        om = 1.0 - pt
        return jnp.sum(a * om * om * ce * keep)

    f1 = focal(conf1, None)
    f2 = focal(conf2, cps)

    lane = jax.lax.broadcasted_iota(i32, (1, 128), 1)
    outv = (jnp.where(lane == 0, n1, 0.0) + jnp.where(lane == 1, n2, 0.0)
            + jnp.where(lane == 2, sl1, 0.0) + jnp.where(lane == 3, sl2, 0.0)
            + jnp.where(lane == 4, f1, 0.0) + jnp.where(lane == 5, f2, 0.0))
    out_ref[...] = outv[None]


def kernel(loc_data, conf_data, priors, targets, im_names):
    B, P, _ = loc_data.shape
    n_obj = targets.shape[1]
    p_pad = ((P + 127) // 128) * 128

    locT = jnp.pad(jnp.transpose(loc_data, (0, 2, 1)),
                   ((0, 0), (0, 0), (0, p_pad - P)))
    confT = jnp.pad(jnp.transpose(conf_data, (0, 2, 1)),
                    ((0, 0), (0, 0), (0, p_pad - P)))
    priT = jnp.transpose(priors, (1, 0))
    pad_col = jnp.array([5.0, 5.0, 0.1, 0.1], jnp.float32)[:, None]
    priT = jnp.concatenate(
        [priT, jnp.broadcast_to(pad_col, (4, p_pad - P))], axis=1)
    tgtT = jnp.transpose(targets, (0, 2, 1))
    tx1 = targets[:, :, 0:1]
    ty1 = targets[:, :, 1:2]
    tx2 = targets[:, :, 2:3]
    ty2 = targets[:, :, 3:4]
    tlab = targets[:, :, 4:5]

    body = functools.partial(_hamloss_body, n_obj=n_obj, p_real=P,
                             p_pad=p_pad)
    out = pl.pallas_call(
        body,
        grid=(B,),
        in_specs=[
            pl.BlockSpec((1, 4, p_pad), lambda b: (b, 0, 0)),
            pl.BlockSpec((1, 2, p_pad), lambda b: (b, 0, 0)),
            pl.BlockSpec((4, p_pad), lambda b: (0, 0)),
            pl.BlockSpec((1, 5, n_obj), lambda b: (b, 0, 0)),
            pl.BlockSpec((1, n_obj, 1), lambda b: (b, 0, 0)),
            pl.BlockSpec((1, n_obj, 1), lambda b: (b, 0, 0)),
            pl.BlockSpec((1, n_obj, 1), lambda b: (b, 0, 0)),
            pl.BlockSpec((1, n_obj, 1), lambda b: (b, 0, 0)),
            pl.BlockSpec((1, n_obj, 1), lambda b: (b, 0, 0)),
        ],
        out_specs=pl.BlockSpec((1, 1, 128), lambda b: (b, 0, 0)),
        out_shape=jax.ShapeDtypeStruct((B, 1, 128), jnp.float32),
        scratch_shapes=[
            pltpu.VMEM((n_obj, p_pad), jnp.float32),
            pltpu.VMEM((1, p_pad), jnp.int32),
        ],
        compiler_params=pltpu.CompilerParams(
            dimension_semantics=("parallel",)),
    )(locT, confT, priT, tgtT, tx1, ty1, tx2, ty2, tlab)

    n1 = jnp.sum(out[:, 0, 0])
    n2 = jnp.sum(out[:, 0, 1])
    sl1 = jnp.sum(out[:, 0, 2])
    sl2 = jnp.sum(out[:, 0, 3])
    f1 = jnp.sum(out[:, 0, 4])
    f2 = jnp.sum(out[:, 0, 5])

    has1 = n1 > 0
    has2 = n2 > 0
    ll1 = sl1 / n1
    ll2 = sl2 / n2
    cl1 = f1 / n1
    cl2 = f2 / n2
    fallback = jnp.asarray(0.0001, jnp.float32)
    loc_loss = jnp.where(has1 & (~has2), ll1,
               jnp.where(has1 & has2, ll1 + ll2,
               jnp.where((~has1) & has2, ll2, fallback)))
    cls_loss = jnp.where(has1 & (~has2), cl1,
               jnp.where(has1 & has2, cl1 + cl2,
               jnp.where((~has1) & has2, cl2, fallback)))
    return (loc_loss, cls_loss)


# speedup vs baseline: 59.6622x; 1.2098x over previous
import functools

import jax
import jax.numpy as jnp
from jax.experimental import pallas as pl
from jax.experimental.pallas import tpu as pltpu

VAR0 = 0.1
VAR1 = 0.2
K = 5
T1 = 0.35
T2 = 0.5
ALPHA = 0.25
GAMMA = 2.0
BETA = 0.11


def _hamloss_body(loc_ref, conf_ref, pri_ref, tgtT_ref,
                  tx1_ref, ty1_ref, tx2_ref, ty2_ref, tlab_ref,
                  out_ref, iou_s, elim_s, *, n_obj, p_real, p_pad):
    f32 = jnp.float32
    i32 = jnp.int32

    loc = loc_ref[0]
    pcx = pri_ref[0:1, :]
    pcy = pri_ref[1:2, :]
    pw = pri_ref[2:3, :]
    ph = pri_ref[3:4, :]
    tgtT = tgtT_ref[0]
    tx1 = tx1_ref[0]
    ty1 = ty1_ref[0]
    tx2 = tx2_ref[0]
    ty2 = ty2_ref[0]

    colI = jax.lax.broadcasted_iota(i32, (n_obj, p_pad), 1)
    rowI = jax.lax.broadcasted_iota(i32, (n_obj, p_pad), 0)
    col1 = jax.lax.broadcasted_iota(i32, (1, p_pad), 1)
    rowV = jax.lax.broadcasted_iota(i32, (n_obj, 1), 0)
    valid = col1 < p_real

    area_t = (tx2 - tx1) * (ty2 - ty1)

    def jaccard(bx1, by1, bx2, by2, area_b):
        ltx = jnp.maximum(tx1, bx1)
        lty = jnp.maximum(ty1, by1)
        rbx = jnp.minimum(tx2, bx2)
        rby = jnp.minimum(ty2, by2)
        iw = jnp.clip(rbx - ltx, 0.0, None)
        ih = jnp.clip(rby - lty, 0.0, None)
        inter = iw * ih
        return inter / (area_t + area_b - inter)

    px1 = pcx - pw * 0.5
    py1 = pcy - ph * 0.5
    px2 = pcx + pw * 0.5
    py2 = pcy + ph * 0.5
    iou0 = jaccard(px1, py1, px2, py2, pw * ph)
    iou0 = jnp.where(valid, iou0, -1.0)
    iou_s[...] = iou0

    bts0 = jnp.max(iou0, axis=0, keepdims=True)
    bti0 = jnp.min(jnp.where(iou0 == bts0, rowI, n_obj), axis=0, keepdims=True)

    elim_s[...] = (~valid).astype(i32)

    bps0 = jnp.max(iou0, axis=1, keepdims=True)
    bpi0 = jnp.min(jnp.where(iou0 == bps0, colI, p_pad), axis=1, keepdims=True)

    def argpick(bps_c, bpi_c, strow, rdead):
        bps_eff = jnp.where(rdead != 0, -1.0, bps_c)
        val = jnp.max(bps_eff)
        j = jnp.min(jnp.where(bps_eff == val, rowV, n_obj))
        i = jnp.sum(jnp.where(rowV == j, bpi_c, 0))
        stale = jnp.sum(jnp.where(rowV == j, strow, 0)) > 0
        return j, i, val, stale

    def greedy_body(t, carry):
        bps, bpi, strow, rdead, js, iss, vals = carry
        j0, i0, val0, stale0 = argpick(bps, bpi, strow, rdead)

        def fix_cond(c):
            return ~c[-1]

        def fix_body(c):
            bps_c, bpi_c, strow_c, j, _, _, _ = c
            rowj = iou_s[pl.ds(j, 1), :]
            rowm = jnp.where(elim_s[...] != 0, -1.0, rowj)
            nv = jnp.max(rowm)
            ni = jnp.min(jnp.where(rowm == nv, col1, p_pad))
            upd = rowV == j
            bps_c = jnp.where(upd, nv, bps_c)
            bpi_c = jnp.where(upd, ni, bpi_c)
            strow_c = jnp.where(upd, 0, strow_c)
            j2, i2, val2, stale2 = argpick(bps_c, bpi_c, strow_c, rdead)
            return (bps_c, bpi_c, strow_c, j2, i2, val2, ~stale2)

        bps, bpi, strow, j, i, val, _ = jax.lax.while_loop(
            fix_cond, fix_body, (bps, bpi, strow, j0, i0, val0, ~stale0))

        elim_s[...] = elim_s[...] | (col1 == i).astype(i32)
        strow = strow | (bpi == i).astype(i32)
        rdead = rdead | (rowV == j).astype(i32)
        upd = rowV == t
        js = jnp.where(upd, j, js)
        iss = jnp.where(upd, i, iss)
        vals = jnp.where(upd, val, vals)
        return (bps, bpi, strow, rdead, js, iss, vals)

    zi = jnp.zeros((n_obj, 1), i32)
    _, _, _, _, js, iss, vals = jax.lax.fori_loop(
        0, n_obj, greedy_body,
        (bps0, bpi0, zi, zi, zi, zi, jnp.zeros((n_obj, 1), jnp.float32)))

    ohp = col1 == iss
    bts_u = jnp.max(jnp.where(ohp, vals, -1e30), axis=0, keepdims=True)
    bti_u = jnp.min(jnp.where(ohp, js, n_obj + 1), axis=0, keepdims=True)
    picked = bts_u > -1e29
    bts = jnp.where(picked, bts_u, bts0)
    bti = jnp.where(picked, bti_u, bti0)

    def gather_rows(idx_row):
        oh = (idx_row == rowI).astype(f32)
        return jax.lax.dot_general(tgtT, oh, (((1,), (0,)), ((), ())),
                                   preferred_element_type=f32)

    g1 = gather_rows(bti)
    conf1 = jnp.where(bts < T1, 0.0, g1[4:5, :])

    def encode(g):
        mx1 = g[0:1, :]
        my1 = g[1:2, :]
        mx2 = g[2:3, :]
        my2 = g[3:4, :]
        e0 = ((mx1 + mx2) * 0.5 - pcx) / (VAR0 * pw)
        e1 = ((my1 + my2) * 0.5 - pcy) / (VAR0 * ph)
        e2 = jnp.log(jnp.clip((mx2 - mx1) / pw, 1e-8, None)) / VAR1
        e3 = jnp.log(jnp.clip((my2 - my1) / ph, 1e-8, None)) / VAR1
        return e0, e1, e2, e3

    enc1 = encode(g1)

    dl0 = loc[0:1, :]
    dl1 = loc[1:2, :]
    dl2 = loc[2:3, :]
    dl3 = loc[3:4, :]
    dcx = pcx + dl0 * (VAR0) * pw
    dcy = pcy + dl1 * (VAR0) * ph
    dw = pw * jnp.exp(dl2 * VAR1)
    dh = ph * jnp.exp(dl3 * VAR1)
    dx1 = dcx - dw * 0.5
    dy1 = dcy - dh * 0.5
    dx2 = dx1 + dw
    dy2 = dy1 + dh
    c_iou = jaccard(dx1, dy1, dx2, dy2, dw * dh)
    c_iou = jnp.where(valid, c_iou, 0.0)

    cbps = jnp.max(c_iou, axis=0, keepdims=True)
    cbpi = jnp.min(jnp.where(c_iou == cbps, rowI, n_obj), axis=0, keepdims=True)

    iou_s[...] = c_iou * (c_iou >= T2).astype(f32)

    ords = []
    tscs = []
    cgs = []
    for _ in range(K):
        cm = iou_s[...]
        tk = jnp.max(cm, axis=1, keepdims=True)
        ok = jnp.min(jnp.where(cm == tk, colI, p_pad), axis=1, keepdims=True)
        mk = colI == ok
        iou_s[...] = jnp.where(mk, -1.0, cm)
        cgs.append(jnp.sum(jnp.where(mk, conf1, 0.0), axis=1, keepdims=True))
        ords.append(ok)
        tscs.append(tk)

    masks = [colI == ok for ok in ords]
    best = None
    for k in range(K):
        hit_k = (cgs[k] < 1.0) & (tscs[k] > 0.0)
        prio_k = jnp.where(hit_k, rowV * K + k, -1)
        c = jnp.where(masks[k], prio_k, -1)
        best = c if best is None else jnp.maximum(best, c)
    bestprio = jnp.max(best, axis=0, keepdims=True)
    tsel = None
    for k in range(K):
        c = jnp.where(masks[k] & ((rowV * K + k) == bestprio), tscs[k], 0.0)
        tsel = c if tsel is None else jnp.maximum(tsel, c)
    ts_win = jnp.max(tsel, axis=0, keepdims=True)
    hitcol = bestprio >= 0
    cps = jnp.where(hitcol, ts_win, 0.0)
    cbpi_f = jnp.where(hitcol, bestprio // K, cbpi)
    g2 = gather_rows(cbpi_f)
    conf2 = jnp.where(cps < T2, -1.0, g2[4:5, :])
    enc2 = encode(g2)

    ign = (bts < T1) & (~(cbps < T2)) & (cps < T2)
    conf1 = jnp.where(ign, -1.0, conf1)

    validf = valid.astype(f32)
    m1 = ((conf1 > 0) & valid).astype(f32)
    m2 = ((conf2 > 0) & valid).astype(f32)
    n1 = jnp.sum(m1)
    n2 = jnp.sum(m2)

    def smooth_l1(enc, m):
        s = jnp.zeros((), f32)
        for r in range(4):
            x = jnp.abs(loc[r:r + 1, :] - enc[r])
            l = jnp.where(x >= BETA, x - 0.5 * BETA, 0.5 * x * x / BETA)
            s = s + jnp.sum(l * m)
        return s

    sl1 = smooth_l1(enc1, m1)
    sl2 = smooth_l1(enc2, m2)

    c0 = conf_ref[0][0:1, :]
    c1 = conf_ref[0][1:2, :]

    def focal(t_row, fiou):
        keep = ((t_row >= 0) & valid).astype(f32)
        t = jnp.maximum(t_row, 0.0)
        x = jnp.where(t >= 0.5, c1, c0)
        ce = jnp.maximum(x, 0.0) - x * t + jnp.log1p(jnp.exp(-jnp.abs(x)))
        a = t * ALPHA + (1.0 - t) * (1.0 - ALPHA)
        if fiou is not None:
            a = a * fiou
        sig = 1.0 / (1.0 + jnp.exp(-x))
        pt = jnp.where(t == 1.0, sig, 1.0 - sig)
        om = 1.0 - pt
        return jnp.sum(a * om * om * ce * keep)

    f1 = focal(conf1, None)
    f2 = focal(conf2, cps)

    lane = jax.lax.broadcasted_iota(i32, (1, 128), 1)
    outv = (jnp.where(lane == 0, n1, 0.0) + jnp.where(lane == 1, n2, 0.0)
            + jnp.where(lane == 2, sl1, 0.0) + jnp.where(lane == 3, sl2, 0.0)
            + jnp.where(lane == 4, f1, 0.0) + jnp.where(lane == 5, f2, 0.0))
    out_ref[...] = outv[None]


def kernel(loc_data, conf_data, priors, targets, im_names):
    B, P, _ = loc_data.shape
    n_obj = targets.shape[1]
    p_pad = ((P + 127) // 128) * 128

    locT = jnp.pad(jnp.transpose(loc_data, (0, 2, 1)),
                   ((0, 0), (0, 0), (0, p_pad - P)))
    confT = jnp.pad(jnp.transpose(conf_data, (0, 2, 1)),
                    ((0, 0), (0, 0), (0, p_pad - P)))
    priT = jnp.transpose(priors, (1, 0))
    pad_col = jnp.array([5.0, 5.0, 0.1, 0.1], jnp.float32)[:, None]
    priT = jnp.concatenate(
        [priT, jnp.broadcast_to(pad_col, (4, p_pad - P))], axis=1)
    tgtT = jnp.transpose(targets, (0, 2, 1))
    tx1 = targets[:, :, 0:1]
    ty1 = targets[:, :, 1:2]
    tx2 = targets[:, :, 2:3]
    ty2 = targets[:, :, 3:4]
    tlab = targets[:, :, 4:5]

    body = functools.partial(_hamloss_body, n_obj=n_obj, p_real=P,
                             p_pad=p_pad)
    out = pl.pallas_call(
        body,
        grid=(B,),
        in_specs=[
            pl.BlockSpec((1, 4, p_pad), lambda b: (b, 0, 0)),
            pl.BlockSpec((1, 2, p_pad), lambda b: (b, 0, 0)),
            pl.BlockSpec((4, p_pad), lambda b: (0, 0)),
            pl.BlockSpec((1, 5, n_obj), lambda b: (b, 0, 0)),
            pl.BlockSpec((1, n_obj, 1), lambda b: (b, 0, 0)),
            pl.BlockSpec((1, n_obj, 1), lambda b: (b, 0, 0)),
            pl.BlockSpec((1, n_obj, 1), lambda b: (b, 0, 0)),
            pl.BlockSpec((1, n_obj, 1), lambda b: (b, 0, 0)),
            pl.BlockSpec((1, n_obj, 1), lambda b: (b, 0, 0)),
        ],
        out_specs=pl.BlockSpec((1, 1, 128), lambda b: (b, 0, 0)),
        out_shape=jax.ShapeDtypeStruct((B, 1, 128), jnp.float32),
        scratch_shapes=[
            pltpu.VMEM((n_obj, p_pad), jnp.float32),
            pltpu.VMEM((1, p_pad), jnp.int32),
        ],
        compiler_params=pltpu.CompilerParams(
            dimension_semantics=("parallel",)),
    )(locT, confT, priT, tgtT, tx1, ty1, tx2, ty2, tlab)

    n1 = jnp.sum(out[:, 0, 0])
    n2 = jnp.sum(out[:, 0, 1])
    sl1 = jnp.sum(out[:, 0, 2])
    sl2 = jnp.sum(out[:, 0, 3])
    f1 = jnp.sum(out[:, 0, 4])
    f2 = jnp.sum(out[:, 0, 5])

    has1 = n1 > 0
    has2 = n2 > 0
    ll1 = sl1 / n1
    ll2 = sl2 / n2
    cl1 = f1 / n1
    cl2 = f2 / n2
    fallback = jnp.asarray(0.0001, jnp.float32)
    loc_loss = jnp.where(has1 & (~has2), ll1,
               jnp.where(has1 & has2, ll1 + ll2,
               jnp.where((~has1) & has2, ll2, fallback)))
    cls_loss = jnp.where(has1 & (~has2), cl1,
               jnp.where(has1 & has2, cl1 + cl2,
               jnp.where((~has1) & has2, cl2, fallback)))
    return (loc_loss, cls_loss)


# speedup vs baseline: 60.3848x; 1.0121x over previous
import functools

import jax
import jax.numpy as jnp
from jax.experimental import pallas as pl
from jax.experimental.pallas import tpu as pltpu

VAR0 = 0.1
VAR1 = 0.2
K = 5
T1 = 0.35
T2 = 0.5
ALPHA = 0.25
GAMMA = 2.0
BETA = 0.11


def _hamloss_body(loc_ref, conf_ref, pri_ref, tgtT_ref,
                  tx1_ref, ty1_ref, tx2_ref, ty2_ref, tlab_ref,
                  out_ref, iou_s, elim_s, *, n_obj, p_real, p_pad):
    f32 = jnp.float32
    i32 = jnp.int32

    loc = loc_ref[0]
    pcx = pri_ref[0:1, :]
    pcy = pri_ref[1:2, :]
    pw = pri_ref[2:3, :]
    ph = pri_ref[3:4, :]
    tgtT = tgtT_ref[0]
    tx1 = tx1_ref[0]
    ty1 = ty1_ref[0]
    tx2 = tx2_ref[0]
    ty2 = ty2_ref[0]

    colI = jax.lax.broadcasted_iota(i32, (n_obj, p_pad), 1)
    rowI = jax.lax.broadcasted_iota(i32, (n_obj, p_pad), 0)
    col1 = jax.lax.broadcasted_iota(i32, (1, p_pad), 1)
    rowV = jax.lax.broadcasted_iota(i32, (n_obj, 1), 0)
    valid = col1 < p_real

    area_t = (tx2 - tx1) * (ty2 - ty1)

    def jaccard(bx1, by1, bx2, by2, area_b):
        ltx = jnp.maximum(tx1, bx1)
        lty = jnp.maximum(ty1, by1)
        rbx = jnp.minimum(tx2, bx2)
        rby = jnp.minimum(ty2, by2)
        iw = jnp.clip(rbx - ltx, 0.0, None)
        ih = jnp.clip(rby - lty, 0.0, None)
        inter = iw * ih
        return inter / (area_t + area_b - inter)

    px1 = pcx - pw * 0.5
    py1 = pcy - ph * 0.5
    px2 = pcx + pw * 0.5
    py2 = pcy + ph * 0.5
    iou0 = jaccard(px1, py1, px2, py2, pw * ph)
    iou0 = jnp.where(valid, iou0, -1.0)
    iou_s[...] = iou0

    bts0 = jnp.max(iou0, axis=0, keepdims=True)
    bti0 = jnp.min(jnp.where(iou0 == bts0, rowI, n_obj), axis=0, keepdims=True)

    elim_s[...] = (~valid).astype(i32)

    bps0 = jnp.max(iou0, axis=1, keepdims=True)
    bpi0 = jnp.min(jnp.where(iou0 == bps0, colI, p_pad), axis=1, keepdims=True)

    def argpick(bps_c, bpi_c, strow, rdead):
        bps_eff = jnp.where(rdead != 0, -1.0, bps_c)
        val = jnp.max(bps_eff)
        j = jnp.min(jnp.where(bps_eff == val, rowV, n_obj))
        i = jnp.sum(jnp.where(rowV == j, bpi_c, 0))
        stale = jnp.sum(jnp.where(rowV == j, strow, 0)) > 0
        return j, i, val, stale

    def greedy_body(t, carry):
        bps, bpi, strow, rdead, js, iss, vals = carry
        j0, i0, val0, stale0 = argpick(bps, bpi, strow, rdead)

        def fix_cond(c):
            return ~c[-1]

        def fix_body(c):
            bps_c, bpi_c, strow_c, j, _, _, _ = c
            rowj = iou_s[pl.ds(j, 1), :]
            rowm = jnp.where(elim_s[...] != 0, -1.0, rowj)
            nv = jnp.max(rowm)
            ni = jnp.min(jnp.where(rowm == nv, col1, p_pad))
            upd = rowV == j
            bps_c = jnp.where(upd, nv, bps_c)
            bpi_c = jnp.where(upd, ni, bpi_c)
            strow_c = jnp.where(upd, 0, strow_c)
            j2, i2, val2, stale2 = argpick(bps_c, bpi_c, strow_c, rdead)
            return (bps_c, bpi_c, strow_c, j2, i2, val2, ~stale2)

        bps, bpi, strow, j, i, val, _ = jax.lax.while_loop(
            fix_cond, fix_body, (bps, bpi, strow, j0, i0, val0, ~stale0))

        elim_s[...] = elim_s[...] | (col1 == i).astype(i32)
        strow = strow | (bpi == i).astype(i32)
        rdead = rdead | (rowV == j).astype(i32)
        upd = rowV == t
        js = jnp.where(upd, j, js)
        iss = jnp.where(upd, i, iss)
        vals = jnp.where(upd, val, vals)
        return (bps, bpi, strow, rdead, js, iss, vals)

    zi = jnp.zeros((n_obj, 1), i32)
    _, _, _, _, js, iss, vals = jax.lax.fori_loop(
        0, n_obj, greedy_body,
        (bps0, bpi0, zi, zi, zi, zi, jnp.zeros((n_obj, 1), jnp.float32)))

    ohp = col1 == iss
    bts_u = jnp.max(jnp.where(ohp, vals, -1e30), axis=0, keepdims=True)
    bti_u = jnp.min(jnp.where(ohp, js, n_obj + 1), axis=0, keepdims=True)
    picked = bts_u > -1e29
    bts = jnp.where(picked, bts_u, bts0)
    bti = jnp.where(picked, bti_u, bti0)

    def gather_rows(idx_row):
        oh = (idx_row == rowI).astype(f32)
        return jax.lax.dot_general(tgtT, oh, (((1,), (0,)), ((), ())),
                                   preferred_element_type=f32)

    g1 = gather_rows(bti)
    conf1 = jnp.where(bts < T1, 0.0, g1[4:5, :])

    def encode(g):
        mx1 = g[0:1, :]
        my1 = g[1:2, :]
        mx2 = g[2:3, :]
        my2 = g[3:4, :]
        e0 = ((mx1 + mx2) * 0.5 - pcx) / (VAR0 * pw)
        e1 = ((my1 + my2) * 0.5 - pcy) / (VAR0 * ph)
        e2 = jnp.log(jnp.clip((mx2 - mx1) / pw, 1e-8, None)) / VAR1
        e3 = jnp.log(jnp.clip((my2 - my1) / ph, 1e-8, None)) / VAR1
        return e0, e1, e2, e3

    enc1 = encode(g1)

    dl0 = loc[0:1, :]
    dl1 = loc[1:2, :]
    dl2 = loc[2:3, :]
    dl3 = loc[3:4, :]
    dcx = pcx + dl0 * (VAR0) * pw
    dcy = pcy + dl1 * (VAR0) * ph
    dw = pw * jnp.exp(dl2 * VAR1)
    dh = ph * jnp.exp(dl3 * VAR1)
    dx1 = dcx - dw * 0.5
    dy1 = dcy - dh * 0.5
    dx2 = dx1 + dw
    dy2 = dy1 + dh
    c_iou = jaccard(dx1, dy1, dx2, dy2, dw * dh)
    c_iou = jnp.where(valid, c_iou, 0.0)

    cbps = jnp.max(c_iou, axis=0, keepdims=True)
    cbpi = jnp.min(jnp.where(c_iou == cbps, rowI, n_obj), axis=0, keepdims=True)

    iou_s[...] = c_iou * (c_iou >= T2).astype(f32)

    ords = []
    tscs = []
    cgs = []
    for _ in range(K):
        cm = iou_s[...]
        tk = jnp.max(cm, axis=1, keepdims=True)
        ok = jnp.min(jnp.where(cm == tk, colI, p_pad), axis=1, keepdims=True)
        mk = colI == ok
        iou_s[...] = jnp.where(mk, -1.0, cm)
        cgs.append(jnp.sum(jnp.where(mk, conf1, 0.0), axis=1, keepdims=True))
        ords.append(ok)
        tscs.append(tk)

    accP = None
    accT = None
    for k in range(K):
        hit_k = (cgs[k] < 1.0) & (tscs[k] > 0.0)
        prio_k = jnp.where(hit_k, rowV * K + k, -1)
        mk = colI == ords[k]
        pk = jnp.where(mk, prio_k, -1)
        if accP is None:
            accP = pk
            accT = jnp.where(mk, tscs[k], 0.0)
        else:
            better = pk > accP
            accP = jnp.where(better, pk, accP)
            accT = jnp.where(better, tscs[k], accT)
    bestprio = jnp.max(accP, axis=0, keepdims=True)
    ts_win = jnp.max(jnp.where(accP == bestprio, accT, 0.0),
                     axis=0, keepdims=True)
    hitcol = bestprio >= 0
    cps = jnp.where(hitcol, ts_win, 0.0)
    cbpi_f = jnp.where(hitcol, bestprio // K, cbpi)
    g2 = gather_rows(cbpi_f)
    conf2 = jnp.where(cps < T2, -1.0, g2[4:5, :])
    enc2 = encode(g2)

    ign = (bts < T1) & (~(cbps < T2)) & (cps < T2)
    conf1 = jnp.where(ign, -1.0, conf1)

    validf = valid.astype(f32)
    m1 = ((conf1 > 0) & valid).astype(f32)
    m2 = ((conf2 > 0) & valid).astype(f32)
    n1 = jnp.sum(m1)
    n2 = jnp.sum(m2)

    def smooth_l1(enc, m):
        s = jnp.zeros((), f32)
        for r in range(4):
            x = jnp.abs(loc[r:r + 1, :] - enc[r])
            l = jnp.where(x >= BETA, x - 0.5 * BETA, 0.5 * x * x / BETA)
            s = s + jnp.sum(l * m)
        return s

    sl1 = smooth_l1(enc1, m1)
    sl2 = smooth_l1(enc2, m2)

    c0 = conf_ref[0][0:1, :]
    c1 = conf_ref[0][1:2, :]

    def focal(t_row, fiou):
        keep = ((t_row >= 0) & valid).astype(f32)
        t = jnp.maximum(t_row, 0.0)
        x = jnp.where(t >= 0.5, c1, c0)
        ce = jnp.maximum(x, 0.0) - x * t + jnp.log1p(jnp.exp(-jnp.abs(x)))
        a = t * ALPHA + (1.0 - t) * (1.0 - ALPHA)
        if fiou is not None:
            a = a * fiou
        sig = 1.0 / (1.0 + jnp.exp(-x))
        pt = jnp.where(t == 1.0, sig, 1.0 - sig)
        om = 1.0 - pt
        return jnp.sum(a * om * om * ce * keep)

    f1 = focal(conf1, None)
    f2 = focal(conf2, cps)

    lane = jax.lax.broadcasted_iota(i32, (1, 128), 1)
    outv = (jnp.where(lane == 0, n1, 0.0) + jnp.where(lane == 1, n2, 0.0)
            + jnp.where(lane == 2, sl1, 0.0) + jnp.where(lane == 3, sl2, 0.0)
            + jnp.where(lane == 4, f1, 0.0) + jnp.where(lane == 5, f2, 0.0))
    out_ref[...] = outv[None]


def kernel(loc_data, conf_data, priors, targets, im_names):
    B, P, _ = loc_data.shape
    n_obj = targets.shape[1]
    p_pad = ((P + 127) // 128) * 128

    locT = jnp.pad(jnp.transpose(loc_data, (0, 2, 1)),
                   ((0, 0), (0, 0), (0, p_pad - P)))
    confT = jnp.pad(jnp.transpose(conf_data, (0, 2, 1)),
                    ((0, 0), (0, 0), (0, p_pad - P)))
    priT = jnp.transpose(priors, (1, 0))
    pad_col = jnp.array([5.0, 5.0, 0.1, 0.1], jnp.float32)[:, None]
    priT = jnp.concatenate(
        [priT, jnp.broadcast_to(pad_col, (4, p_pad - P))], axis=1)
    tgtT = jnp.transpose(targets, (0, 2, 1))
    tx1 = targets[:, :, 0:1]
    ty1 = targets[:, :, 1:2]
    tx2 = targets[:, :, 2:3]
    ty2 = targets[:, :, 3:4]
    tlab = targets[:, :, 4:5]

    body = functools.partial(_hamloss_body, n_obj=n_obj, p_real=P,
                             p_pad=p_pad)
    out = pl.pallas_call(
        body,
        grid=(B,),
        in_specs=[
            pl.BlockSpec((1, 4, p_pad), lambda b: (b, 0, 0)),
            pl.BlockSpec((1, 2, p_pad), lambda b: (b, 0, 0)),
            pl.BlockSpec((4, p_pad), lambda b: (0, 0)),
            pl.BlockSpec((1, 5, n_obj), lambda b: (b, 0, 0)),
            pl.BlockSpec((1, n_obj, 1), lambda b: (b, 0, 0)),
            pl.BlockSpec((1, n_obj, 1), lambda b: (b, 0, 0)),
            pl.BlockSpec((1, n_obj, 1), lambda b: (b, 0, 0)),
            pl.BlockSpec((1, n_obj, 1), lambda b: (b, 0, 0)),
            pl.BlockSpec((1, n_obj, 1), lambda b: (b, 0, 0)),
        ],
        out_specs=pl.BlockSpec((1, 1, 128), lambda b: (b, 0, 0)),
        out_shape=jax.ShapeDtypeStruct((B, 1, 128), jnp.float32),
        scratch_shapes=[
            pltpu.VMEM((n_obj, p_pad), jnp.float32),
            pltpu.VMEM((1, p_pad), jnp.int32),
        ],
        compiler_params=pltpu.CompilerParams(
            dimension_semantics=("parallel",)),
    )(locT, confT, priT, tgtT, tx1, ty1, tx2, ty2, tlab)

    n1 = jnp.sum(out[:, 0, 0])
    n2 = jnp.sum(out[:, 0, 1])
    sl1 = jnp.sum(out[:, 0, 2])
    sl2 = jnp.sum(out[:, 0, 3])
    f1 = jnp.sum(out[:, 0, 4])
    f2 = jnp.sum(out[:, 0, 5])

    has1 = n1 > 0
    has2 = n2 > 0
    ll1 = sl1 / n1
    ll2 = sl2 / n2
    cl1 = f1 / n1
    cl2 = f2 / n2
    fallback = jnp.asarray(0.0001, jnp.float32)
    loc_loss = jnp.where(has1 & (~has2), ll1,
               jnp.where(has1 & has2, ll1 + ll2,
               jnp.where((~has1) & has2, ll2, fallback)))
    cls_loss = jnp.where(has1 & (~has2), cl1,
               jnp.where(has1 & has2, cl1 + cl2,
               jnp.where((~has1) & has2, cl2, fallback)))
    return (loc_loss, cls_loss)
